# baseline (speedup 1.0000x reference)
import functools

import jax
import jax.numpy as jnp
import numpy as np
from jax.experimental import pallas as pl
from jax.experimental.pallas import tpu as pltpu
from jax.experimental.pallas import tpu_sc as plsc

N = 10000
E = 640000
HID = 16
NODE_IN = 10
AGENT_DIM = 5
RE = E // 8
RN = N // 8
BB = 800
GRID_E = RE // BB

_INTERPRET = False


def _kron8(w):
    return jnp.kron(jnp.eye(8, dtype=jnp.float32),
                    w.astype(jnp.float32)).astype(jnp.bfloat16)


def _t8(b):
    return jnp.tile(b.astype(jnp.float32), 8).reshape(1, -1)


def _wspec(a):
    n = a.ndim
    return pl.BlockSpec(a.shape, lambda i, _n=n: (0,) * _n)


def _espec(minor):
    return pl.BlockSpec((BB, minor), lambda i: (i, 0))


def _dot(a, b):
    return jnp.dot(a.astype(jnp.bfloat16), b,
                   preferred_element_type=jnp.float32)


def _ln(x, M, s, b):
    mu = _dot(x, M)
    xc = x - mu
    var = _dot(xc * xc, M)
    return xc * jax.lax.rsqrt(var + 1e-5) * s + b


def _prep_body(nfp, afg, pe, WA, WB, WP, bt, Wq, bqt, Wk, bkt, Wv, bvt,
               h_o, q_o, k_o, v_o):
    h = (_dot(nfp[...], WA[...]) + _dot(afg[...], WB[...])
         + _dot(pe[...], WP[...]) + bt[...])
    h_o[...] = h
    q_o[...] = _dot(h, Wq[...]) + bqt[...]
    k_o[...] = _dot(h, Wk[...]) + bkt[...]
    v_o[...] = _dot(h, Wv[...]) + bvt[...]


def _c0_body(ef8, qd, ks, vs, R, bet, We, bewt, SX, Woe, boet,
             W1, b1t, W2, b2t, s1, bb1, s2, bb2, M,
             e1_o, wm_o, wx_o):
    e = _dot(ef8[...], R[...]) + bet[...]
    ew = _dot(e, We[...]) + bewt[...]
    score = qd[...] * ks[...] * ew * 0.5
    wex = jnp.exp(jnp.clip(_dot(score, SX[...]), -5.0, 5.0))
    gate = jax.nn.sigmoid(ew)
    wm_o[...] = wex * vs[...] * gate
    wx_o[...] = wex
    x = _ln(e + _dot(score, Woe[...]) + boet[...], M[...], s1[...], bb1[...])
    f = _dot(jax.nn.relu(_dot(x, W1[...]) + b1t[...]), W2[...]) + b2t[...]
    e1_o[...] = _ln(x + f, M[...], s2[...], bb2[...])


def _n0_body(h, hs0, hs1, hs2, hs3, ds0, ds1, ds2, ds3, Wo, bot,
             W1, b1t, W2, b2t, s1, bb1, s2, bb2, M, Wq, bqt, Wk, bkt,
             q_o, k_o):
    hagg = ((hs0[...] + hs1[...]) + (hs2[...] + hs3[...])) / (
        (ds0[...] + ds1[...]) + (ds2[...] + ds3[...]) + 1e-9)
    hn = _dot(hagg, Wo[...]) + bot[...]
    x = _ln(h[...] + hn, M[...], s1[...], bb1[...])
    f = _dot(jax.nn.relu(_dot(x, W1[...]) + b1t[...]), W2[...]) + b2t[...]
    x = _ln(x + f, M[...], s2[...], bb2[...])
    q_o[...] = _dot(x, Wq[...]) + bqt[...]
    k_o[...] = _dot(x, Wk[...]) + bkt[...]


def _c1_body(e1, qd, ks, We, bewt, Woe, boet, W1, b1t, W2, b2t,
             s1, bb1, s2, bb2, M, Wout, bout,
             esc_o):
    e = e1[...]
    ew = _dot(e, We[...]) + bewt[...]
    score = qd[...] * ks[...] * ew * 0.5
    x = _ln(e + _dot(score, Woe[...]) + boet[...], M[...], s1[...], bb1[...])
    f = _dot(jax.nn.relu(_dot(x, W1[...]) + b1t[...]), W2[...]) + b2t[...]
    e2 = _ln(x + f, M[...], s2[...], bb2[...])
    esc_o[...] = jax.nn.sigmoid(_dot(e2, Wout[...]) + bout[...])


def _tc_full(body, n_out):
    def run(*args):
        out_shape = tuple(jax.ShapeDtypeStruct((RN, 128), jnp.float32)
                          for _ in range(n_out))
        return pl.pallas_call(body, out_shape=out_shape,
                              interpret=_INTERPRET)(*args)
    return run


def _tc_edge(body, in_minors, out_minors, rows):
    def run(*args):
        n_data = len(in_minors)
        in_specs = [_espec(m) for m in in_minors]
        in_specs += [_wspec(a) for a in args[n_data:]]
        out_specs = tuple(_espec(m) for m in out_minors)
        out_shape = tuple(jax.ShapeDtypeStruct((rows, m), jnp.float32)
                          for m in out_minors)
        return pl.pallas_call(body, grid=(rows // BB,), in_specs=in_specs,
                              out_specs=out_specs, out_shape=out_shape,
                              interpret=_INTERPRET)(*args)
    return run


_NW = 32


def _sc_mesh():
    return plsc.VectorSubcoreMesh(core_axis_name="c", subcore_axis_name="s")


def _sc_gather(tables, idxs, tmap, total, ch, base=0):
    n_t = len(tables)
    n_u = len(idxs)
    per_w = total // _NW
    n_ch = per_w // ch
    out_type = tuple(jax.ShapeDtypeStruct((total, HID), jnp.float32)
                     for _ in range(n_t))
    scratch = ([pltpu.VMEM((2, ch), jnp.int32) for _ in range(n_u)]
               + [pltpu.VMEM((2, ch, HID), jnp.float32) for _ in range(n_t)]
               + [pltpu.SemaphoreType.DMA] * 3)

    def body(*refs):
        t_refs = refs[:n_t]
        i_refs = refs[n_t:n_t + n_u]
        o_refs = refs[n_t + n_u:2 * n_t + n_u]
        iv = refs[2 * n_t + n_u:2 * n_t + 2 * n_u]
        rv = refs[2 * n_t + 2 * n_u:3 * n_t + 2 * n_u]
        sem_i, sem_g, sem_w = refs[-3:]
        wid = jax.lax.axis_index("s") * 2 + jax.lax.axis_index("c")
        base0 = wid * per_w

        def start_idx(i, slot):
            for u in range(n_u):
                pltpu.async_copy(i_refs[u].at[pl.ds(base + base0 + i * ch,
                                                    ch)],
                                 iv[u].at[slot], sem_i)

        def wait_idx():
            for u in range(n_u):
                pltpu.make_async_copy(i_refs[u].at[pl.ds(0, ch)],
                                      iv[u].at[0], sem_i).wait()

        def start_gather(i, slot):
            for t in range(n_t):
                pltpu.async_copy(t_refs[t].at[iv[tmap[t]].at[slot]],
                                 rv[t].at[slot], sem_g)

        def wait_gather():
            for t in range(n_t):
                pltpu.make_async_copy(t_refs[t].at[iv[tmap[t]].at[0]],
                                      rv[t].at[0], sem_g).wait()

        def start_wb(i, slot):
            for t in range(n_t):
                pltpu.async_copy(rv[t].at[slot],
                                 o_refs[t].at[pl.ds(base0 + i * ch, ch)],
                                 sem_w)

        def wait_wb():
            for t in range(n_t):
                pltpu.make_async_copy(rv[t].at[0],
                                      o_refs[t].at[pl.ds(0, ch)], sem_w).wait()

        if n_ch < 2:
            for i in range(n_ch):
                start_idx(i, 0)
                wait_idx()
                start_gather(i, 0)
                wait_gather()
                start_wb(i, 0)
                wait_wb()
            return

        def half(i, slot):
            @pl.when(i > 0)
            def _():
                wait_gather()
                start_wb(i - 1, 1 - slot)

            wait_idx()

            @pl.when(i >= 2)
            def _():
                wait_wb()

            start_gather(i, slot)

            @pl.when(i + 1 < n_ch)
            def _():
                start_idx(i + 1, 1 - slot)

        def it2(j, carry):
            half(2 * j, 0)
            half(2 * j + 1, 1)
            return carry

        start_idx(0, 0)
        jax.lax.fori_loop(0, n_ch // 2, it2, 0)
        wait_gather()
        start_wb(n_ch - 1, (n_ch - 1) % 2)
        wait_wb()
        wait_wb()

    return pl.kernel(
        body, out_type=out_type, mesh=_sc_mesh(), scratch_types=scratch,
        compiler_params=pltpu.CompilerParams(use_tc_tiling_on_sc=False),
    )(*tables, *idxs)


def _gather_agent(afp, agent_index):
    npad = 10240
    idx = jnp.pad(agent_index, (0, npad - N))
    (out,) = _sc_gather([afp], [idx], [0], npad, 320)
    return out[:N]


def _sc_scatter(dst, wmsg, wex, zeros, total, base=0):
    ch = 1000
    per_w = total // _NW
    n_ch = per_w // ch
    rps = N // 16
    out_type = (jax.ShapeDtypeStruct((2, N, HID), jnp.float32),
                jax.ShapeDtypeStruct((2, N, HID), jnp.float32))
    scratch = [pltpu.VMEM((2, ch), jnp.int32),
               pltpu.VMEM((2, ch, HID), jnp.float32),
               pltpu.VMEM((2, ch, HID), jnp.float32),
               pltpu.VMEM_SHARED((N, HID), jnp.float32),
               pltpu.VMEM_SHARED((N, HID), jnp.float32),
               pltpu.SemaphoreType.DMA,
               pltpu.SemaphoreType.DMA]

    def body(dst_ref, wm_ref, wx_ref, z_ref, hs_out, ds_out,
             idx_v, wm_v, wx_v, hsh, dsh, sem_l, sem_s):
        c = jax.lax.axis_index("c")
        s = jax.lax.axis_index("s")
        wid = s * 2 + c
        pltpu.sync_copy(z_ref, hsh.at[pl.ds(s * rps, rps)])
        pltpu.sync_copy(z_ref, dsh.at[pl.ds(s * rps, rps)])
        plsc.subcore_barrier()
        base0 = wid * per_w

        def start_load(i, slot):
            loc = base0 + i * ch
            pltpu.async_copy(dst_ref.at[pl.ds(base + loc, ch)],
                             idx_v.at[slot], sem_l)
            pltpu.async_copy(wm_ref.at[pl.ds(loc, ch)], wm_v.at[slot], sem_l)
            pltpu.async_copy(wx_ref.at[pl.ds(loc, ch)], wx_v.at[slot], sem_l)

        def wait_load():
            pltpu.make_async_copy(dst_ref.at[pl.ds(0, ch)], idx_v.at[0],
                                  sem_l).wait()
            pltpu.make_async_copy(wm_ref.at[pl.ds(0, ch)], wm_v.at[0],
                                  sem_l).wait()
            pltpu.make_async_copy(wx_ref.at[pl.ds(0, ch)], wx_v.at[0],
                                  sem_l).wait()

        def start_scat(slot):
            pltpu.async_copy(wm_v.at[slot], hsh.at[idx_v.at[slot]], sem_s,
                             add=True)
            pltpu.async_copy(wx_v.at[slot], dsh.at[idx_v.at[slot]], sem_s,
                             add=True)

        def wait_scat():
            pltpu.make_async_copy(wm_v.at[0], hsh.at[idx_v.at[0]],
                                  sem_s).wait()
            pltpu.make_async_copy(wx_v.at[0], dsh.at[idx_v.at[0]],
                                  sem_s).wait()

        def half(i, slot):
            wait_load()
            start_scat(slot)

            @pl.when(jnp.logical_and(i >= 1, i + 1 < n_ch))
            def _():
                wait_scat()

            @pl.when(i + 1 < n_ch)
            def _():
                start_load(i + 1, 1 - slot)

        def it2(j, carry):
            half(2 * j, 0)
            half(2 * j + 1, 1)
            return carry

        start_load(0, 0)
        jax.lax.fori_loop(0, n_ch // 2, it2, 0)
        wait_scat()
        wait_scat()
        plsc.subcore_barrier()
        pltpu.sync_copy(hsh.at[pl.ds(s * rps, rps)],
                        hs_out.at[c, pl.ds(s * rps, rps)])
        pltpu.sync_copy(dsh.at[pl.ds(s * rps, rps)],
                        ds_out.at[c, pl.ds(s * rps, rps)])

    hsP, dsP = pl.kernel(
        body, out_type=out_type, mesh=_sc_mesh(), scratch_types=scratch,
        compiler_params=pltpu.CompilerParams(use_tc_tiling_on_sc=False),
    )(dst, wmsg, wex, zeros)
    return hsP[0], hsP[1], dsP[0], dsP[1]


def kernel(node_features, edge_features, params, edge_index, agent_index):
    p = params
    f32 = jnp.float32
    src = edge_index[0]
    dst = edge_index[1]

    M = _kron8(jnp.full((HID, HID), 1.0 / HID, f32))
    SX = _kron8(jnp.kron(jnp.eye(4, dtype=f32), jnp.ones((4, 4), f32)))

    WinA = jnp.zeros((HID, HID), f32).at[:NODE_IN].set(p['W_in'][:NODE_IN])
    WinB = jnp.zeros((HID, HID), f32).at[:AGENT_DIM].set(p['W_in'][NODE_IN:])
    nfp = jnp.pad(node_features, ((0, 0), (0, HID - NODE_IN)))
    afp = jnp.pad(p['agent_features'], ((0, 0), (0, HID - AGENT_DIM)))
    afg = _gather_agent(afp, agent_index)
    bt = _t8(p['b_in'] + p['b_pe'])

    h8, q08, k08, v08 = _tc_full(_prep_body, 4)(
        nfp.reshape(RN, 128), afg.reshape(RN, 128),
        p['positional_embedding'].reshape(RN, 128),
        _kron8(WinA), _kron8(WinB), _kron8(p['W_pe']), bt,
        _kron8(p['Wq'][0]), _t8(p['bq'][0]),
        _kron8(p['Wk'][0]), _t8(p['bk'][0]),
        _kron8(p['Wv'][0]), _t8(p['bv'][0]))

    NS = 2
    ES = E // NS
    RS = RE // NS
    qt0 = q08.reshape(N, HID)
    kt0 = k08.reshape(N, HID)
    vt0 = v08.reshape(N, HID)

    R = jnp.kron(jnp.eye(8, dtype=f32),
                 p['W_e_in'].astype(f32)).astype(jnp.bfloat16)
    c0_w = (R, _t8(p['b_e_in']),
            _kron8(p['We'][0]), _t8(p['be'][0]), SX,
            _kron8(p['Woe'][0]), _t8(p['boe'][0]),
            _kron8(p['Wef1'][0]), _t8(p['bef1'][0]),
            _kron8(p['Wef2'][0]), _t8(p['bef2'][0]),
            _t8(p['lne1_s'][0]), _t8(p['lne1_b'][0]),
            _t8(p['lne2_s'][0]), _t8(p['lne2_b'][0]), M)
    ef8 = edge_features.reshape(RE, 8)
    zeros = jnp.zeros((N // 16, HID), f32)

    g0 = [_sc_gather([qt0, kt0, vt0], [dst, src], [0, 1, 1], ES, 1000,
                     base=s * ES) for s in range(NS)]
    c0 = [_tc_edge(_c0_body, [8, 128, 128, 128], [128, 128, 128], RS)(
        ef8[s * RS:(s + 1) * RS], g0[s][0].reshape(RS, 128),
        g0[s][1].reshape(RS, 128), g0[s][2].reshape(RS, 128), *c0_w)
        for s in range(NS)]
    sc = [_sc_scatter(dst, c0[s][1].reshape(ES, HID),
                      c0[s][2].reshape(ES, HID), zeros, ES, base=s * ES)
          for s in range(NS)]

    q18, k18 = _tc_full(_n0_body, 2)(
        h8,
        sc[0][0].reshape(RN, 128), sc[0][1].reshape(RN, 128),
        sc[1][0].reshape(RN, 128), sc[1][1].reshape(RN, 128),
        sc[0][2].reshape(RN, 128), sc[0][3].reshape(RN, 128),
        sc[1][2].reshape(RN, 128), sc[1][3].reshape(RN, 128),
        _kron8(p['Wo'][0]), _t8(p['bo'][0]),
        _kron8(p['Wf1'][0]), _t8(p['bf1'][0]),
        _kron8(p['Wf2'][0]), _t8(p['bf2'][0]),
        _t8(p['ln1_s'][0]), _t8(p['ln1_b'][0]),
        _t8(p['ln2_s'][0]), _t8(p['ln2_b'][0]), M,
        _kron8(p['Wq'][1]), _t8(p['bq'][1]),
        _kron8(p['Wk'][1]), _t8(p['bk'][1]))

    Wout8 = jnp.kron(jnp.eye(8, dtype=f32),
                     p['W_out'].astype(f32)).astype(jnp.bfloat16)
    bout = jnp.tile(p['b_out'].astype(f32), 8).reshape(1, 8)
    c1_w = (_kron8(p['We'][1]), _t8(p['be'][1]),
            _kron8(p['Woe'][1]), _t8(p['boe'][1]),
            _kron8(p['Wef1'][1]), _t8(p['bef1'][1]),
            _kron8(p['Wef2'][1]), _t8(p['bef2'][1]),
            _t8(p['lne1_s'][1]), _t8(p['lne1_b'][1]),
            _t8(p['lne2_s'][1]), _t8(p['lne2_b'][1]), M,
            Wout8, bout)
    qt1 = q18.reshape(N, HID)
    kt1 = k18.reshape(N, HID)

    g1 = [_sc_gather([qt1, kt1], [dst, src], [0, 1], ES, 1000, base=s * ES)
          for s in range(NS)]
    esc = [_tc_edge(_c1_body, [128, 128, 128], [8], RS)(
        c0[s][0], g1[s][0].reshape(RS, 128), g1[s][1].reshape(RS, 128),
        *c1_w)[0]
        for s in range(NS)]

    return jnp.concatenate(esc, axis=0).reshape(E)


# speedup vs baseline: 1.0618x; 1.0618x over previous
import functools

import jax
import jax.numpy as jnp
import numpy as np
from jax.experimental import pallas as pl
from jax.experimental.pallas import tpu as pltpu
from jax.experimental.pallas import tpu_sc as plsc

N = 10000
E = 640000
HID = 16
NODE_IN = 10
AGENT_DIM = 5
RE = E // 8
RN = N // 8
BB = 800
GRID_E = RE // BB

_INTERPRET = False


def _kron8(w):
    return jnp.kron(jnp.eye(8, dtype=jnp.float32),
                    w.astype(jnp.float32)).astype(jnp.bfloat16)


def _t8(b):
    return jnp.tile(b.astype(jnp.float32), 8).reshape(1, -1)


def _wspec(a):
    n = a.ndim
    return pl.BlockSpec(a.shape, lambda i, _n=n: (0,) * _n)


def _espec(minor):
    return pl.BlockSpec((BB, minor), lambda i: (i, 0))


def _dot(a, b):
    return jnp.dot(a.astype(jnp.bfloat16), b,
                   preferred_element_type=jnp.float32)


def _ln(x, M, s, b):
    mu = _dot(x, M)
    xc = x - mu
    var = _dot(xc * xc, M)
    return xc * jax.lax.rsqrt(var + 1e-5) * s + b


def _prep_body(nfp, afg, pe, WA, WB, WP, bt, Wq, bqt, Wk, bkt, Wv, bvt,
               h_o, q_o, k_o, v_o):
    h = (_dot(nfp[...], WA[...]) + _dot(afg[...], WB[...])
         + _dot(pe[...], WP[...]) + bt[...])
    h_o[...] = h
    q_o[...] = _dot(h, Wq[...]) + bqt[...]
    k_o[...] = _dot(h, Wk[...]) + bkt[...]
    v_o[...] = _dot(h, Wv[...]) + bvt[...]


def _c0_body(ef8, qd, ks, vs, R, bet, We, bewt, SX, Woe, boet,
             W1, b1t, W2, b2t, s1, bb1, s2, bb2, M,
             e1_o, wm_o, wx_o):
    e = _dot(ef8[...], R[...]) + bet[...]
    ew = _dot(e, We[...]) + bewt[...]
    score = qd[...] * ks[...] * ew
    wex = jnp.exp(jnp.clip(_dot(score, SX[...]), -5.0, 5.0))
    gate = jax.nn.sigmoid(ew)
    wm_o[...] = wex * vs[...] * gate
    wx_o[...] = wex
    x = _ln(e + _dot(score, Woe[...]) + boet[...], M[...], s1[...], bb1[...])
    f = _dot(jax.nn.relu(_dot(x, W1[...]) + b1t[...]), W2[...]) + b2t[...]
    e1_o[...] = _ln(x + f, M[...], s2[...], bb2[...])


def _n0_body(h, hs0, hs1, hs2, hs3, ds0, ds1, ds2, ds3, Wo, bot,
             W1, b1t, W2, b2t, s1, bb1, s2, bb2, M, Wq, bqt, Wk, bkt,
             q_o, k_o):
    hagg = ((hs0[...] + hs1[...]) + (hs2[...] + hs3[...])) / (
        (ds0[...] + ds1[...]) + (ds2[...] + ds3[...]) + 1e-9)
    hn = _dot(hagg, Wo[...]) + bot[...]
    x = _ln(h[...] + hn, M[...], s1[...], bb1[...])
    f = _dot(jax.nn.relu(_dot(x, W1[...]) + b1t[...]), W2[...]) + b2t[...]
    x = _ln(x + f, M[...], s2[...], bb2[...])
    q_o[...] = _dot(x, Wq[...]) + bqt[...]
    k_o[...] = _dot(x, Wk[...]) + bkt[...]


def _c1_body(e1, qd, ks, We, bewt, Woe, boet, W1, b1t, W2, b2t,
             s1, bb1, s2, bb2, M, Wout, bout,
             esc_o):
    e = e1[...]
    ew = _dot(e, We[...]) + bewt[...]
    score = qd[...] * ks[...] * ew
    x = _ln(e + _dot(score, Woe[...]) + boet[...], M[...], s1[...], bb1[...])
    f = _dot(jax.nn.relu(_dot(x, W1[...]) + b1t[...]), W2[...]) + b2t[...]
    e2 = _ln(x + f, M[...], s2[...], bb2[...])
    esc_o[...] = jax.nn.sigmoid(_dot(e2, Wout[...]) + bout[...])


def _tc_full(body, n_out):
    def run(*args):
        out_shape = tuple(jax.ShapeDtypeStruct((RN, 128), jnp.float32)
                          for _ in range(n_out))
        return pl.pallas_call(body, out_shape=out_shape,
                              interpret=_INTERPRET)(*args)
    return run


def _tc_edge(body, in_minors, out_minors, rows):
    def run(*args):
        n_data = len(in_minors)
        in_specs = [_espec(m) for m in in_minors]
        in_specs += [_wspec(a) for a in args[n_data:]]
        out_specs = tuple(_espec(m) for m in out_minors)
        out_shape = tuple(jax.ShapeDtypeStruct((rows, m), jnp.float32)
                          for m in out_minors)
        return pl.pallas_call(body, grid=(rows // BB,), in_specs=in_specs,
                              out_specs=out_specs, out_shape=out_shape,
                              interpret=_INTERPRET)(*args)
    return run


_NW = 32


def _sc_mesh():
    return plsc.VectorSubcoreMesh(core_axis_name="c", subcore_axis_name="s")


def _sc_gather(tables, idxs, tmap, total, ch, base=0):
    n_t = len(tables)
    n_u = len(idxs)
    per_w = total // _NW
    n_ch = per_w // ch
    out_type = tuple(jax.ShapeDtypeStruct((total, HID), jnp.float32)
                     for _ in range(n_t))
    scratch = ([pltpu.VMEM((2, ch), jnp.int32) for _ in range(n_u)]
               + [pltpu.VMEM((2, ch, HID), jnp.float32) for _ in range(n_t)]
               + [pltpu.SemaphoreType.DMA] * 3)

    def body(*refs):
        t_refs = refs[:n_t]
        i_refs = refs[n_t:n_t + n_u]
        o_refs = refs[n_t + n_u:2 * n_t + n_u]
        iv = refs[2 * n_t + n_u:2 * n_t + 2 * n_u]
        rv = refs[2 * n_t + 2 * n_u:3 * n_t + 2 * n_u]
        sem_i, sem_g, sem_w = refs[-3:]
        wid = jax.lax.axis_index("s") * 2 + jax.lax.axis_index("c")
        base0 = wid * per_w

        def start_idx(i, slot):
            for u in range(n_u):
                pltpu.async_copy(i_refs[u].at[pl.ds(base + base0 + i * ch,
                                                    ch)],
                                 iv[u].at[slot], sem_i)

        def wait_idx():
            for u in range(n_u):
                pltpu.make_async_copy(i_refs[u].at[pl.ds(0, ch)],
                                      iv[u].at[0], sem_i).wait()

        def start_gather(i, slot):
            for t in range(n_t):
                pltpu.async_copy(t_refs[t].at[iv[tmap[t]].at[slot]],
                                 rv[t].at[slot], sem_g)

        def wait_gather():
            for t in range(n_t):
                pltpu.make_async_copy(t_refs[t].at[iv[tmap[t]].at[0]],
                                      rv[t].at[0], sem_g).wait()

        def start_wb(i, slot):
            for t in range(n_t):
                pltpu.async_copy(rv[t].at[slot],
                                 o_refs[t].at[pl.ds(base0 + i * ch, ch)],
                                 sem_w)

        def wait_wb():
            for t in range(n_t):
                pltpu.make_async_copy(rv[t].at[0],
                                      o_refs[t].at[pl.ds(0, ch)], sem_w).wait()

        if n_ch < 2:
            for i in range(n_ch):
                start_idx(i, 0)
                wait_idx()
                start_gather(i, 0)
                wait_gather()
                start_wb(i, 0)
                wait_wb()
            return

        def half(i, slot):
            @pl.when(i > 0)
            def _():
                wait_gather()
                start_wb(i - 1, 1 - slot)

            wait_idx()

            @pl.when(i >= 2)
            def _():
                wait_wb()

            start_gather(i, slot)

            @pl.when(i + 1 < n_ch)
            def _():
                start_idx(i + 1, 1 - slot)

        def it2(j, carry):
            half(2 * j, 0)
            half(2 * j + 1, 1)
            return carry

        start_idx(0, 0)
        jax.lax.fori_loop(0, n_ch // 2, it2, 0)
        wait_gather()
        start_wb(n_ch - 1, (n_ch - 1) % 2)
        wait_wb()
        wait_wb()

    return pl.kernel(
        body, out_type=out_type, mesh=_sc_mesh(), scratch_types=scratch,
        compiler_params=pltpu.CompilerParams(use_tc_tiling_on_sc=False),
    )(*tables, *idxs)


def _gather_agent(afp, agent_index):
    npad = 10240
    idx = jnp.pad(agent_index, (0, npad - N))
    (out,) = _sc_gather([afp], [idx], [0], npad, 320)
    return out[:N]


def _sc_scatter(dst, wmsg, wex, zeros, total, base=0):
    ch = 1000
    per_w = total // _NW
    n_ch = per_w // ch
    rps = N // 16
    out_type = (jax.ShapeDtypeStruct((2, N, HID), jnp.float32),
                jax.ShapeDtypeStruct((2, N, HID), jnp.float32))
    scratch = [pltpu.VMEM((2, ch), jnp.int32),
               pltpu.VMEM((2, ch, HID), jnp.float32),
               pltpu.VMEM((2, ch, HID), jnp.float32),
               pltpu.VMEM_SHARED((N, HID), jnp.float32),
               pltpu.VMEM_SHARED((N, HID), jnp.float32),
               pltpu.SemaphoreType.DMA,
               pltpu.SemaphoreType.DMA]

    def body(dst_ref, wm_ref, wx_ref, z_ref, hs_out, ds_out,
             idx_v, wm_v, wx_v, hsh, dsh, sem_l, sem_s):
        c = jax.lax.axis_index("c")
        s = jax.lax.axis_index("s")
        wid = s * 2 + c
        pltpu.sync_copy(z_ref, hsh.at[pl.ds(s * rps, rps)])
        pltpu.sync_copy(z_ref, dsh.at[pl.ds(s * rps, rps)])
        plsc.subcore_barrier()
        base0 = wid * per_w

        def start_load(i, slot):
            loc = base0 + i * ch
            pltpu.async_copy(dst_ref.at[pl.ds(base + loc, ch)],
                             idx_v.at[slot], sem_l)
            pltpu.async_copy(wm_ref.at[pl.ds(loc, ch)], wm_v.at[slot], sem_l)
            pltpu.async_copy(wx_ref.at[pl.ds(loc, ch)], wx_v.at[slot], sem_l)

        def wait_load():
            pltpu.make_async_copy(dst_ref.at[pl.ds(0, ch)], idx_v.at[0],
                                  sem_l).wait()
            pltpu.make_async_copy(wm_ref.at[pl.ds(0, ch)], wm_v.at[0],
                                  sem_l).wait()
            pltpu.make_async_copy(wx_ref.at[pl.ds(0, ch)], wx_v.at[0],
                                  sem_l).wait()

        def start_scat(slot):
            pltpu.async_copy(wm_v.at[slot], hsh.at[idx_v.at[slot]], sem_s,
                             add=True)
            pltpu.async_copy(wx_v.at[slot], dsh.at[idx_v.at[slot]], sem_s,
                             add=True)

        def wait_scat():
            pltpu.make_async_copy(wm_v.at[0], hsh.at[idx_v.at[0]],
                                  sem_s).wait()
            pltpu.make_async_copy(wx_v.at[0], dsh.at[idx_v.at[0]],
                                  sem_s).wait()

        def half(i, slot):
            wait_load()
            start_scat(slot)

            @pl.when(jnp.logical_and(i >= 1, i + 1 < n_ch))
            def _():
                wait_scat()

            @pl.when(i + 1 < n_ch)
            def _():
                start_load(i + 1, 1 - slot)

        def it2(j, carry):
            half(2 * j, 0)
            half(2 * j + 1, 1)
            return carry

        start_load(0, 0)
        jax.lax.fori_loop(0, n_ch // 2, it2, 0)
        wait_scat()
        wait_scat()
        plsc.subcore_barrier()
        pltpu.sync_copy(hsh.at[pl.ds(s * rps, rps)],
                        hs_out.at[c, pl.ds(s * rps, rps)])
        pltpu.sync_copy(dsh.at[pl.ds(s * rps, rps)],
                        ds_out.at[c, pl.ds(s * rps, rps)])

    hsP, dsP = pl.kernel(
        body, out_type=out_type, mesh=_sc_mesh(), scratch_types=scratch,
        compiler_params=pltpu.CompilerParams(use_tc_tiling_on_sc=False),
    )(dst, wmsg, wex, zeros)
    return hsP[0], hsP[1], dsP[0], dsP[1]


def kernel(node_features, edge_features, params, edge_index, agent_index):
    p = params
    f32 = jnp.float32
    src = edge_index[0]
    dst = edge_index[1]

    M = _kron8(jnp.full((HID, HID), 1.0 / HID, f32))
    SX = _kron8(jnp.kron(jnp.eye(4, dtype=f32), jnp.ones((4, 4), f32)))

    WinA = jnp.zeros((HID, HID), f32).at[:NODE_IN].set(p['W_in'][:NODE_IN])
    WinB = jnp.zeros((HID, HID), f32).at[:AGENT_DIM].set(p['W_in'][NODE_IN:])
    nfp = jnp.pad(node_features, ((0, 0), (0, HID - NODE_IN)))
    afp = jnp.pad(p['agent_features'], ((0, 0), (0, HID - AGENT_DIM)))
    afg = _gather_agent(afp, agent_index)
    bt = _t8(p['b_in'] + p['b_pe'])

    h8, q08, k08, v08 = _tc_full(_prep_body, 4)(
        nfp.reshape(RN, 128), afg.reshape(RN, 128),
        p['positional_embedding'].reshape(RN, 128),
        _kron8(WinA), _kron8(WinB), _kron8(p['W_pe']), bt,
        _kron8(p['Wq'][0] * 0.5), _t8(p['bq'][0] * 0.5),
        _kron8(p['Wk'][0]), _t8(p['bk'][0]),
        _kron8(p['Wv'][0]), _t8(p['bv'][0]))

    NS = 1
    ES = E // NS
    RS = RE // NS
    qt0 = q08.reshape(N, HID)
    kt0 = k08.reshape(N, HID)
    vt0 = v08.reshape(N, HID)

    R = jnp.kron(jnp.eye(8, dtype=f32),
                 p['W_e_in'].astype(f32)).astype(jnp.bfloat16)
    c0_w = (R, _t8(p['b_e_in']),
            _kron8(p['We'][0]), _t8(p['be'][0]), SX,
            _kron8(p['Woe'][0]), _t8(p['boe'][0]),
            _kron8(p['Wef1'][0]), _t8(p['bef1'][0]),
            _kron8(p['Wef2'][0]), _t8(p['bef2'][0]),
            _t8(p['lne1_s'][0]), _t8(p['lne1_b'][0]),
            _t8(p['lne2_s'][0]), _t8(p['lne2_b'][0]), M)
    ef8 = edge_features.reshape(RE, 8)
    zeros = jnp.zeros((N // 16, HID), f32)

    g0 = [_sc_gather([qt0, kt0, vt0], [dst, src], [0, 1, 1], ES, 1000,
                     base=s * ES) for s in range(NS)]
    c0 = [_tc_edge(_c0_body, [8, 128, 128, 128], [128, 128, 128], RS)(
        ef8[s * RS:(s + 1) * RS], g0[s][0].reshape(RS, 128),
        g0[s][1].reshape(RS, 128), g0[s][2].reshape(RS, 128), *c0_w)
        for s in range(NS)]
    sc = [_sc_scatter(dst, c0[s][1].reshape(ES, HID),
                      c0[s][2].reshape(ES, HID), zeros, ES, base=s * ES)
          for s in range(NS)]

    zRN = jnp.zeros((RN, 128), f32)
    hs_args = [sc[s][i].reshape(RN, 128) for s in range(NS) for i in (0, 1)]
    ds_args = [sc[s][i].reshape(RN, 128) for s in range(NS) for i in (2, 3)]
    hs_args += [zRN] * (4 - len(hs_args))
    ds_args += [zRN] * (4 - len(ds_args))
    q18, k18 = _tc_full(_n0_body, 2)(
        h8, *hs_args, *ds_args,
        _kron8(p['Wo'][0]), _t8(p['bo'][0]),
        _kron8(p['Wf1'][0]), _t8(p['bf1'][0]),
        _kron8(p['Wf2'][0]), _t8(p['bf2'][0]),
        _t8(p['ln1_s'][0]), _t8(p['ln1_b'][0]),
        _t8(p['ln2_s'][0]), _t8(p['ln2_b'][0]), M,
        _kron8(p['Wq'][1] * 0.5), _t8(p['bq'][1] * 0.5),
        _kron8(p['Wk'][1]), _t8(p['bk'][1]))

    Wout8 = jnp.kron(jnp.eye(8, dtype=f32),
                     p['W_out'].astype(f32)).astype(jnp.bfloat16)
    bout = jnp.tile(p['b_out'].astype(f32), 8).reshape(1, 8)
    c1_w = (_kron8(p['We'][1]), _t8(p['be'][1]),
            _kron8(p['Woe'][1]), _t8(p['boe'][1]),
            _kron8(p['Wef1'][1]), _t8(p['bef1'][1]),
            _kron8(p['Wef2'][1]), _t8(p['bef2'][1]),
            _t8(p['lne1_s'][1]), _t8(p['lne1_b'][1]),
            _t8(p['lne2_s'][1]), _t8(p['lne2_b'][1]), M,
            Wout8, bout)
    qt1 = q18.reshape(N, HID)
    kt1 = k18.reshape(N, HID)

    g1 = [_sc_gather([qt1, kt1], [dst, src], [0, 1], ES, 1000, base=s * ES)
          for s in range(NS)]
    esc = [_tc_edge(_c1_body, [128, 128, 128], [8], RS)(
        c0[s][0], g1[s][0].reshape(RS, 128), g1[s][1].reshape(RS, 128),
        *c1_w)[0]
        for s in range(NS)]

    return jnp.concatenate(esc, axis=0).reshape(E)


# speedup vs baseline: 1.1116x; 1.0469x over previous
import functools

import jax
import jax.numpy as jnp
import numpy as np
from jax.experimental import pallas as pl
from jax.experimental.pallas import tpu as pltpu
from jax.experimental.pallas import tpu_sc as plsc

N = 10000
E = 640000
HID = 16
NODE_IN = 10
AGENT_DIM = 5
RE = E // 8
RN = N // 8
BB = 800
GRID_E = RE // BB

_INTERPRET = False


def _kron8(w):
    return jnp.kron(jnp.eye(8, dtype=jnp.float32),
                    w.astype(jnp.float32)).astype(jnp.bfloat16)


def _t8(b):
    return jnp.tile(b.astype(jnp.float32), 8).reshape(1, -1)


def _wspec(a):
    n = a.ndim
    return pl.BlockSpec(a.shape, lambda i, _n=n: (0,) * _n)


def _espec(minor):
    return pl.BlockSpec((BB, minor), lambda i: (i, 0))


def _dot(a, b):
    return jnp.dot(a.astype(jnp.bfloat16), b,
                   preferred_element_type=jnp.float32)


def _ln(x, M, s, b):
    mu = _dot(x, M)
    xc = x - mu
    var = _dot(xc * xc, M)
    return xc * jax.lax.rsqrt(var + 1e-5) * s + b


def _prep_body(nfp, afg, pe, WA, WB, WP, bt, Wq, bqt, Wk, bkt, Wv, bvt,
               h_o, q_o, k_o, v_o):
    h = (_dot(nfp[...], WA[...]) + _dot(afg[...], WB[...])
         + _dot(pe[...], WP[...]) + bt[...])
    h_o[...] = h
    q_o[...] = _dot(h, Wq[...]) + bqt[...]
    k_o[...] = _dot(h, Wk[...]) + bkt[...]
    v_o[...] = _dot(h, Wv[...]) + bvt[...]


def _c0_body(ef8, qk, vs, R, bet, We, bewt, SX, Woe, boet,
             W1, b1t, W2, b2t, s1, bb1, s2, bb2, M,
             e1_o, wm_o, wx_o):
    e = _dot(ef8[...], R[...]) + bet[...]
    ew = _dot(e, We[...]) + bewt[...]
    score = qk[...] * ew
    wex = jnp.exp(jnp.clip(_dot(score, SX[...]), -5.0, 5.0))
    gate = jax.nn.sigmoid(ew)
    wm_o[...] = wex * vs[...] * gate
    wx_o[...] = wex
    x = _ln(e + _dot(score, Woe[...]) + boet[...], M[...], s1[...], bb1[...])
    f = _dot(jax.nn.relu(_dot(x, W1[...]) + b1t[...]), W2[...]) + b2t[...]
    e1_o[...] = _ln(x + f, M[...], s2[...], bb2[...])


def _n0_body(h, hs0, hs1, hs2, hs3, ds0, ds1, ds2, ds3, Wo, bot,
             W1, b1t, W2, b2t, s1, bb1, s2, bb2, M, Wq, bqt, Wk, bkt,
             q_o, k_o):
    hagg = ((hs0[...] + hs1[...]) + (hs2[...] + hs3[...])) / (
        (ds0[...] + ds1[...]) + (ds2[...] + ds3[...]) + 1e-9)
    hn = _dot(hagg, Wo[...]) + bot[...]
    x = _ln(h[...] + hn, M[...], s1[...], bb1[...])
    f = _dot(jax.nn.relu(_dot(x, W1[...]) + b1t[...]), W2[...]) + b2t[...]
    x = _ln(x + f, M[...], s2[...], bb2[...])
    q_o[...] = _dot(x, Wq[...]) + bqt[...]
    k_o[...] = _dot(x, Wk[...]) + bkt[...]


def _c1_body(e1, qk, We, bewt, Woe, boet, W1, b1t, W2, b2t,
             s1, bb1, s2, bb2, M, Wout, bout,
             esc_o):
    e = e1[...]
    ew = _dot(e, We[...]) + bewt[...]
    score = qk[...] * ew
    x = _ln(e + _dot(score, Woe[...]) + boet[...], M[...], s1[...], bb1[...])
    f = _dot(jax.nn.relu(_dot(x, W1[...]) + b1t[...]), W2[...]) + b2t[...]
    e2 = _ln(x + f, M[...], s2[...], bb2[...])
    esc_o[...] = jax.nn.sigmoid(_dot(e2, Wout[...]) + bout[...])


def _tc_full(body, n_out):
    def run(*args):
        out_shape = tuple(jax.ShapeDtypeStruct((RN, 128), jnp.float32)
                          for _ in range(n_out))
        return pl.pallas_call(body, out_shape=out_shape,
                              interpret=_INTERPRET)(*args)
    return run


def _tc_edge(body, in_minors, out_minors, rows):
    def run(*args):
        n_data = len(in_minors)
        in_specs = [_espec(m) for m in in_minors]
        in_specs += [_wspec(a) for a in args[n_data:]]
        out_specs = tuple(_espec(m) for m in out_minors)
        out_shape = tuple(jax.ShapeDtypeStruct((rows, m), jnp.float32)
                          for m in out_minors)
        return pl.pallas_call(body, grid=(rows // BB,), in_specs=in_specs,
                              out_specs=out_specs, out_shape=out_shape,
                              interpret=_INTERPRET)(*args)
    return run


_NW = 32


def _sc_mesh():
    return plsc.VectorSubcoreMesh(core_axis_name="c", subcore_axis_name="s")


def _sc_gather(tables, idxs, tmap, total, ch, base=0, fold01=False):
    n_t = len(tables)
    n_u = len(idxs)
    per_w = total // _NW
    n_ch = per_w // ch
    n_o = n_t - 1 if fold01 else n_t
    omap = ([0] + list(range(2, n_t))) if fold01 else list(range(n_t))
    out_type = tuple(jax.ShapeDtypeStruct((total, HID), jnp.float32)
                     for _ in range(n_o))
    scratch = ([pltpu.VMEM((2, ch), jnp.int32) for _ in range(n_u)]
               + [pltpu.VMEM((2, ch, HID), jnp.float32) for _ in range(n_t)]
               + [pltpu.SemaphoreType.DMA] * 3)

    def body(*refs):
        t_refs = refs[:n_t]
        i_refs = refs[n_t:n_t + n_u]
        o_refs = refs[n_t + n_u:n_t + n_u + n_o]
        iv = refs[n_t + n_u + n_o:n_t + 2 * n_u + n_o]
        rv = refs[n_t + 2 * n_u + n_o:2 * n_t + 2 * n_u + n_o]
        sem_i, sem_g, sem_w = refs[-3:]
        wid = jax.lax.axis_index("s") * 2 + jax.lax.axis_index("c")
        base0 = wid * per_w

        def start_idx(i, slot):
            for u in range(n_u):
                pltpu.async_copy(i_refs[u].at[pl.ds(base + base0 + i * ch,
                                                    ch)],
                                 iv[u].at[slot], sem_i)

        def wait_idx():
            for u in range(n_u):
                pltpu.make_async_copy(i_refs[u].at[pl.ds(0, ch)],
                                      iv[u].at[0], sem_i).wait()

        def start_gather(i, slot):
            for t in range(n_t):
                pltpu.async_copy(t_refs[t].at[iv[tmap[t]].at[slot]],
                                 rv[t].at[slot], sem_g)

        def wait_gather():
            for t in range(n_t):
                pltpu.make_async_copy(t_refs[t].at[iv[tmap[t]].at[0]],
                                      rv[t].at[0], sem_g).wait()

        def mul01(slot):
            if not fold01:
                return

            def mbody(r, c):
                for u2 in range(4):
                    i2 = r * 4 + u2
                    rv[0][slot, i2, :] = rv[0][slot, i2, :] * rv[1][slot, i2, :]
                return c

            jax.lax.fori_loop(0, ch // 4, mbody, 0)

        def start_wb(i, slot):
            for j in range(n_o):
                pltpu.async_copy(rv[omap[j]].at[slot],
                                 o_refs[j].at[pl.ds(base0 + i * ch, ch)],
                                 sem_w)

        def wait_wb():
            for j in range(n_o):
                pltpu.make_async_copy(rv[omap[j]].at[0],
                                      o_refs[j].at[pl.ds(0, ch)], sem_w).wait()

        if n_ch < 2:
            for i in range(n_ch):
                start_idx(i, 0)
                wait_idx()
                start_gather(i, 0)
                wait_gather()
                mul01(0)
                start_wb(i, 0)
                wait_wb()
            return

        def half(i, slot):
            @pl.when(i > 0)
            def _():
                wait_gather()

            wait_idx()

            @pl.when(i >= 2)
            def _():
                wait_wb()

            start_gather(i, slot)

            @pl.when(i > 0)
            def _():
                mul01(1 - slot)
                start_wb(i - 1, 1 - slot)

            @pl.when(i + 1 < n_ch)
            def _():
                start_idx(i + 1, 1 - slot)

        def it2(j, carry):
            half(2 * j, 0)
            half(2 * j + 1, 1)
            return carry

        start_idx(0, 0)
        jax.lax.fori_loop(0, n_ch // 2, it2, 0)
        wait_gather()
        mul01((n_ch - 1) % 2)
        start_wb(n_ch - 1, (n_ch - 1) % 2)
        wait_wb()
        wait_wb()

    return pl.kernel(
        body, out_type=out_type, mesh=_sc_mesh(), scratch_types=scratch,
        compiler_params=pltpu.CompilerParams(use_tc_tiling_on_sc=False),
    )(*tables, *idxs)


def _gather_agent(afp, agent_index):
    npad = 10240
    idx = jnp.pad(agent_index, (0, npad - N))
    (out,) = _sc_gather([afp], [idx], [0], npad, 320)
    return out[:N]


def _sc_scatter(dst, wmsg, wex, zeros, total, base=0):
    ch = 1000
    per_w = total // _NW
    n_ch = per_w // ch
    rps = N // 16
    out_type = (jax.ShapeDtypeStruct((2, N, HID), jnp.float32),
                jax.ShapeDtypeStruct((2, N, HID), jnp.float32))
    scratch = [pltpu.VMEM((2, ch), jnp.int32),
               pltpu.VMEM((2, ch, HID), jnp.float32),
               pltpu.VMEM((2, ch, HID), jnp.float32),
               pltpu.VMEM_SHARED((N, HID), jnp.float32),
               pltpu.VMEM_SHARED((N, HID), jnp.float32),
               pltpu.SemaphoreType.DMA,
               pltpu.SemaphoreType.DMA]

    def body(dst_ref, wm_ref, wx_ref, z_ref, hs_out, ds_out,
             idx_v, wm_v, wx_v, hsh, dsh, sem_l, sem_s):
        c = jax.lax.axis_index("c")
        s = jax.lax.axis_index("s")
        wid = s * 2 + c
        pltpu.sync_copy(z_ref, hsh.at[pl.ds(s * rps, rps)])
        pltpu.sync_copy(z_ref, dsh.at[pl.ds(s * rps, rps)])
        plsc.subcore_barrier()
        base0 = wid * per_w

        def start_load(i, slot):
            loc = base0 + i * ch
            pltpu.async_copy(dst_ref.at[pl.ds(base + loc, ch)],
                             idx_v.at[slot], sem_l)
            pltpu.async_copy(wm_ref.at[pl.ds(loc, ch)], wm_v.at[slot], sem_l)
            pltpu.async_copy(wx_ref.at[pl.ds(loc, ch)], wx_v.at[slot], sem_l)

        def wait_load():
            pltpu.make_async_copy(dst_ref.at[pl.ds(0, ch)], idx_v.at[0],
                                  sem_l).wait()
            pltpu.make_async_copy(wm_ref.at[pl.ds(0, ch)], wm_v.at[0],
                                  sem_l).wait()
            pltpu.make_async_copy(wx_ref.at[pl.ds(0, ch)], wx_v.at[0],
                                  sem_l).wait()

        def start_scat(slot):
            pltpu.async_copy(wm_v.at[slot], hsh.at[idx_v.at[slot]], sem_s,
                             add=True)
            pltpu.async_copy(wx_v.at[slot], dsh.at[idx_v.at[slot]], sem_s,
                             add=True)

        def wait_scat():
            pltpu.make_async_copy(wm_v.at[0], hsh.at[idx_v.at[0]],
                                  sem_s).wait()
            pltpu.make_async_copy(wx_v.at[0], dsh.at[idx_v.at[0]],
                                  sem_s).wait()

        def half(i, slot):
            wait_load()
            start_scat(slot)

            @pl.when(jnp.logical_and(i >= 1, i + 1 < n_ch))
            def _():
                wait_scat()

            @pl.when(i + 1 < n_ch)
            def _():
                start_load(i + 1, 1 - slot)

        def it2(j, carry):
            half(2 * j, 0)
            half(2 * j + 1, 1)
            return carry

        start_load(0, 0)
        jax.lax.fori_loop(0, n_ch // 2, it2, 0)
        wait_scat()
        wait_scat()
        plsc.subcore_barrier()
        pltpu.sync_copy(hsh.at[pl.ds(s * rps, rps)],
                        hs_out.at[c, pl.ds(s * rps, rps)])
        pltpu.sync_copy(dsh.at[pl.ds(s * rps, rps)],
                        ds_out.at[c, pl.ds(s * rps, rps)])

    hsP, dsP = pl.kernel(
        body, out_type=out_type, mesh=_sc_mesh(), scratch_types=scratch,
        compiler_params=pltpu.CompilerParams(use_tc_tiling_on_sc=False),
    )(dst, wmsg, wex, zeros)
    return hsP[0], hsP[1], dsP[0], dsP[1]


def kernel(node_features, edge_features, params, edge_index, agent_index):
    p = params
    f32 = jnp.float32
    src = edge_index[0]
    dst = edge_index[1]

    M = _kron8(jnp.full((HID, HID), 1.0 / HID, f32))
    SX = _kron8(jnp.kron(jnp.eye(4, dtype=f32), jnp.ones((4, 4), f32)))

    WinA = jnp.zeros((HID, HID), f32).at[:NODE_IN].set(p['W_in'][:NODE_IN])
    WinB = jnp.zeros((HID, HID), f32).at[:AGENT_DIM].set(p['W_in'][NODE_IN:])
    nfp = jnp.pad(node_features, ((0, 0), (0, HID - NODE_IN)))
    afp = jnp.pad(p['agent_features'], ((0, 0), (0, HID - AGENT_DIM)))
    afg = _gather_agent(afp, agent_index)
    bt = _t8(p['b_in'] + p['b_pe'])

    h8, q08, k08, v08 = _tc_full(_prep_body, 4)(
        nfp.reshape(RN, 128), afg.reshape(RN, 128),
        p['positional_embedding'].reshape(RN, 128),
        _kron8(WinA), _kron8(WinB), _kron8(p['W_pe']), bt,
        _kron8(p['Wq'][0] * 0.5), _t8(p['bq'][0] * 0.5),
        _kron8(p['Wk'][0]), _t8(p['bk'][0]),
        _kron8(p['Wv'][0]), _t8(p['bv'][0]))

    NS = 1
    ES = E // NS
    RS = RE // NS
    qt0 = q08.reshape(N, HID)
    kt0 = k08.reshape(N, HID)
    vt0 = v08.reshape(N, HID)

    R = jnp.kron(jnp.eye(8, dtype=f32),
                 p['W_e_in'].astype(f32)).astype(jnp.bfloat16)
    c0_w = (R, _t8(p['b_e_in']),
            _kron8(p['We'][0]), _t8(p['be'][0]), SX,
            _kron8(p['Woe'][0]), _t8(p['boe'][0]),
            _kron8(p['Wef1'][0]), _t8(p['bef1'][0]),
            _kron8(p['Wef2'][0]), _t8(p['bef2'][0]),
            _t8(p['lne1_s'][0]), _t8(p['lne1_b'][0]),
            _t8(p['lne2_s'][0]), _t8(p['lne2_b'][0]), M)
    ef8 = edge_features.reshape(RE, 8)
    zeros = jnp.zeros((N // 16, HID), f32)

    g0 = [_sc_gather([qt0, kt0, vt0], [dst, src], [0, 1, 1], ES, 1000,
                     base=s * ES, fold01=True) for s in range(NS)]
    c0 = [_tc_edge(_c0_body, [8, 128, 128], [128, 128, 128], RS)(
        ef8[s * RS:(s + 1) * RS], g0[s][0].reshape(RS, 128),
        g0[s][1].reshape(RS, 128), *c0_w)
        for s in range(NS)]
    sc = [_sc_scatter(dst, c0[s][1].reshape(ES, HID),
                      c0[s][2].reshape(ES, HID), zeros, ES, base=s * ES)
          for s in range(NS)]

    zRN = jnp.zeros((RN, 128), f32)
    hs_args = [sc[s][i].reshape(RN, 128) for s in range(NS) for i in (0, 1)]
    ds_args = [sc[s][i].reshape(RN, 128) for s in range(NS) for i in (2, 3)]
    hs_args += [zRN] * (4 - len(hs_args))
    ds_args += [zRN] * (4 - len(ds_args))
    q18, k18 = _tc_full(_n0_body, 2)(
        h8, *hs_args, *ds_args,
        _kron8(p['Wo'][0]), _t8(p['bo'][0]),
        _kron8(p['Wf1'][0]), _t8(p['bf1'][0]),
        _kron8(p['Wf2'][0]), _t8(p['bf2'][0]),
        _t8(p['ln1_s'][0]), _t8(p['ln1_b'][0]),
        _t8(p['ln2_s'][0]), _t8(p['ln2_b'][0]), M,
        _kron8(p['Wq'][1] * 0.5), _t8(p['bq'][1] * 0.5),
        _kron8(p['Wk'][1]), _t8(p['bk'][1]))

    Wout8 = jnp.kron(jnp.eye(8, dtype=f32),
                     p['W_out'].astype(f32)).astype(jnp.bfloat16)
    bout = jnp.tile(p['b_out'].astype(f32), 8).reshape(1, 8)
    c1_w = (_kron8(p['We'][1]), _t8(p['be'][1]),
            _kron8(p['Woe'][1]), _t8(p['boe'][1]),
            _kron8(p['Wef1'][1]), _t8(p['bef1'][1]),
            _kron8(p['Wef2'][1]), _t8(p['bef2'][1]),
            _t8(p['lne1_s'][1]), _t8(p['lne1_b'][1]),
            _t8(p['lne2_s'][1]), _t8(p['lne2_b'][1]), M,
            Wout8, bout)
    qt1 = q18.reshape(N, HID)
    kt1 = k18.reshape(N, HID)

    g1 = [_sc_gather([qt1, kt1], [dst, src], [0, 1], ES, 1000, base=s * ES,
                     fold01=True)
          for s in range(NS)]
    esc = [_tc_edge(_c1_body, [128, 128], [8], RS)(
        c0[s][0], g1[s][0].reshape(RS, 128), *c1_w)[0]
        for s in range(NS)]

    return jnp.concatenate(esc, axis=0).reshape(E)


# speedup vs baseline: 1.3102x; 1.1786x over previous
import functools

import jax
import jax.numpy as jnp
import numpy as np
from jax.experimental import pallas as pl
from jax.experimental.pallas import tpu as pltpu
from jax.experimental.pallas import tpu_sc as plsc

N = 10000
E = 640000
HID = 16
NODE_IN = 10
AGENT_DIM = 5
RE = E // 8
RN = N // 8
BB = 1600
GRID_E = RE // BB

_INTERPRET = False


def _kron8(w):
    return jnp.kron(jnp.eye(8, dtype=jnp.float32),
                    w.astype(jnp.float32)).astype(jnp.bfloat16)


def _t8(b):
    return jnp.tile(b.astype(jnp.float32), 8).reshape(1, -1)


def _wspec(a):
    n = a.ndim
    return pl.BlockSpec(a.shape, lambda i, _n=n: (0,) * _n)


def _espec(minor):
    return pl.BlockSpec((BB, minor), lambda i: (i, 0))


def _dot(a, b):
    return jnp.dot(a.astype(jnp.bfloat16), b,
                   preferred_element_type=jnp.float32)


def _ln(x, M, s, b):
    mu = _dot(x, M)
    xc = x - mu
    var = _dot(xc * xc, M)
    return xc * jax.lax.rsqrt(var + 1e-5) * s + b


def _prep_body(nfp, afg, pe, WA, WB, WP, bt, Wq, bqt, Wk, bkt, Wv, bvt,
               h_o, q_o, k_o, v_o):
    h = (_dot(nfp[...], WA[...]) + _dot(afg[...], WB[...])
         + _dot(pe[...], WP[...]) + bt[...])
    h_o[...] = h
    q_o[...] = _dot(h, Wq[...]) + bqt[...]
    k_o[...] = _dot(h, Wk[...]) + bkt[...]
    v_o[...] = _dot(h, Wv[...]) + bvt[...]


def _c0_body(ef8, qk, vs, R, bet, We, bewt, SX, Woe, boet,
             W1, b1t, W2, b2t, s1, bb1, s2, bb2, M,
             e1_o, wm_o, wx_o):
    e = _dot(ef8[...], R[...]) + bet[...]
    ew = _dot(e, We[...]) + bewt[...]
    score = qk[...] * ew
    wex = jnp.exp(jnp.clip(_dot(score, SX[...]), -5.0, 5.0))
    gate = jax.nn.sigmoid(ew)
    wm_o[...] = wex * vs[...] * gate
    wx_o[...] = wex
    x = _ln(e + _dot(score, Woe[...]) + boet[...], M[...], s1[...], bb1[...])
    f = _dot(jax.nn.relu(_dot(x, W1[...]) + b1t[...]), W2[...]) + b2t[...]
    e1_o[...] = _ln(x + f, M[...], s2[...], bb2[...]).astype(jnp.bfloat16)


def _n0_body(h, hs0, hs1, hs2, hs3, ds0, ds1, ds2, ds3, Wo, bot,
             W1, b1t, W2, b2t, s1, bb1, s2, bb2, M, Wq, bqt, Wk, bkt,
             q_o, k_o):
    hagg = ((hs0[...] + hs1[...]) + (hs2[...] + hs3[...])) / (
        (ds0[...] + ds1[...]) + (ds2[...] + ds3[...]) + 1e-9)
    hn = _dot(hagg, Wo[...]) + bot[...]
    x = _ln(h[...] + hn, M[...], s1[...], bb1[...])
    f = _dot(jax.nn.relu(_dot(x, W1[...]) + b1t[...]), W2[...]) + b2t[...]
    x = _ln(x + f, M[...], s2[...], bb2[...])
    q_o[...] = _dot(x, Wq[...]) + bqt[...]
    k_o[...] = _dot(x, Wk[...]) + bkt[...]


def _c1_body(e1, qk, We, bewt, Woe, boet, W1, b1t, W2, b2t,
             s1, bb1, s2, bb2, M, Wout, bout,
             esc_o):
    e = e1[...].astype(jnp.float32)
    ew = _dot(e, We[...]) + bewt[...]
    score = qk[...] * ew
    x = _ln(e + _dot(score, Woe[...]) + boet[...], M[...], s1[...], bb1[...])
    f = _dot(jax.nn.relu(_dot(x, W1[...]) + b1t[...]), W2[...]) + b2t[...]
    e2 = _ln(x + f, M[...], s2[...], bb2[...])
    esc_o[...] = jax.nn.sigmoid(_dot(e2, Wout[...]) + bout[...])


def _tc_full(body, n_out):
    def run(*args):
        out_shape = tuple(jax.ShapeDtypeStruct((RN, 128), jnp.float32)
                          for _ in range(n_out))
        return pl.pallas_call(body, out_shape=out_shape,
                              interpret=_INTERPRET)(*args)
    return run


def _tc_edge(body, in_minors, out_minors, rows, out_dtypes=None):
    def run(*args):
        n_data = len(in_minors)
        in_specs = [_espec(m) for m in in_minors]
        in_specs += [_wspec(a) for a in args[n_data:]]
        out_specs = tuple(_espec(m) for m in out_minors)
        dts = out_dtypes or [jnp.float32] * len(out_minors)
        out_shape = tuple(jax.ShapeDtypeStruct((rows, m), dt)
                          for m, dt in zip(out_minors, dts))
        return pl.pallas_call(body, grid=(rows // BB,), in_specs=in_specs,
                              out_specs=out_specs, out_shape=out_shape,
                              interpret=_INTERPRET)(*args)
    return run


_NW = 32


def _sc_mesh():
    return plsc.VectorSubcoreMesh(core_axis_name="c", subcore_axis_name="s")


def _sc_gather(tables, idxs, tmap, total, ch, base=0, fold01=False):
    n_t = len(tables)
    n_u = len(idxs)
    per_w = total // _NW
    n_ch = per_w // ch
    n_o = n_t - 1 if fold01 else n_t
    omap = ([0] + list(range(2, n_t))) if fold01 else list(range(n_t))
    out_type = tuple(jax.ShapeDtypeStruct((total, HID), jnp.float32)
                     for _ in range(n_o))
    scratch = ([pltpu.VMEM((2, ch), jnp.int32) for _ in range(n_u)]
               + [pltpu.VMEM((2, ch, HID), jnp.float32) for _ in range(n_t)]
               + [pltpu.SemaphoreType.DMA] * 3)

    def body(*refs):
        t_refs = refs[:n_t]
        i_refs = refs[n_t:n_t + n_u]
        o_refs = refs[n_t + n_u:n_t + n_u + n_o]
        iv = refs[n_t + n_u + n_o:n_t + 2 * n_u + n_o]
        rv = refs[n_t + 2 * n_u + n_o:2 * n_t + 2 * n_u + n_o]
        sem_i, sem_g, sem_w = refs[-3:]
        wid = jax.lax.axis_index("s") * 2 + jax.lax.axis_index("c")
        base0 = wid * per_w

        def start_idx(i, slot):
            for u in range(n_u):
                pltpu.async_copy(i_refs[u].at[pl.ds(base + base0 + i * ch,
                                                    ch)],
                                 iv[u].at[slot], sem_i)

        def wait_idx():
            for u in range(n_u):
                pltpu.make_async_copy(i_refs[u].at[pl.ds(0, ch)],
                                      iv[u].at[0], sem_i).wait()

        def start_gather(i, slot):
            for t in range(n_t):
                pltpu.async_copy(t_refs[t].at[iv[tmap[t]].at[slot]],
                                 rv[t].at[slot], sem_g)

        def wait_gather():
            for t in range(n_t):
                pltpu.make_async_copy(t_refs[t].at[iv[tmap[t]].at[0]],
                                      rv[t].at[0], sem_g).wait()

        def mul01(slot):
            if not fold01:
                return

            def mbody(r, c):
                for u2 in range(4):
                    i2 = r * 4 + u2
                    rv[0][slot, i2, :] = rv[0][slot, i2, :] * rv[1][slot, i2, :]
                return c

            jax.lax.fori_loop(0, ch // 4, mbody, 0)

        def start_wb(i, slot):
            for j in range(n_o):
                pltpu.async_copy(rv[omap[j]].at[slot],
                                 o_refs[j].at[pl.ds(base0 + i * ch, ch)],
                                 sem_w)

        def wait_wb():
            for j in range(n_o):
                pltpu.make_async_copy(rv[omap[j]].at[0],
                                      o_refs[j].at[pl.ds(0, ch)], sem_w).wait()

        if n_ch < 2:
            for i in range(n_ch):
                start_idx(i, 0)
                wait_idx()
                start_gather(i, 0)
                wait_gather()
                mul01(0)
                start_wb(i, 0)
                wait_wb()
            return

        def half(i, slot):
            @pl.when(i > 0)
            def _():
                wait_gather()

            wait_idx()

            @pl.when(i >= 2)
            def _():
                wait_wb()

            start_gather(i, slot)

            @pl.when(i > 0)
            def _():
                mul01(1 - slot)
                start_wb(i - 1, 1 - slot)

            @pl.when(i + 1 < n_ch)
            def _():
                start_idx(i + 1, 1 - slot)

        def it2(j, carry):
            half(2 * j, 0)
            half(2 * j + 1, 1)
            return carry

        start_idx(0, 0)
        jax.lax.fori_loop(0, n_ch // 2, it2, 0)
        wait_gather()
        mul01((n_ch - 1) % 2)
        start_wb(n_ch - 1, (n_ch - 1) % 2)
        wait_wb()
        wait_wb()

    return pl.kernel(
        body, out_type=out_type, mesh=_sc_mesh(), scratch_types=scratch,
        compiler_params=pltpu.CompilerParams(use_tc_tiling_on_sc=False),
    )(*tables, *idxs)


def _gather_agent(afp, agent_index):
    npad = 10240
    idx = jnp.pad(agent_index, (0, npad - N))
    (out,) = _sc_gather([afp], [idx], [0], npad, 320)
    return out[:N]


def _sc_scatter(dst, wmsg, wex, zeros, total, base=0):
    ch = 1000
    per_w = total // _NW
    n_ch = per_w // ch
    rps = N // 16
    out_type = (jax.ShapeDtypeStruct((2, N, HID), jnp.float32),
                jax.ShapeDtypeStruct((2, N, HID), jnp.float32))
    scratch = [pltpu.VMEM((2, ch), jnp.int32),
               pltpu.VMEM((2, ch, HID), jnp.float32),
               pltpu.VMEM((2, ch, HID), jnp.float32),
               pltpu.VMEM_SHARED((N, HID), jnp.float32),
               pltpu.VMEM_SHARED((N, HID), jnp.float32),
               pltpu.SemaphoreType.DMA,
               pltpu.SemaphoreType.DMA]

    def body(dst_ref, wm_ref, wx_ref, z_ref, hs_out, ds_out,
             idx_v, wm_v, wx_v, hsh, dsh, sem_l, sem_s):
        c = jax.lax.axis_index("c")
        s = jax.lax.axis_index("s")
        wid = s * 2 + c
        pltpu.sync_copy(z_ref, hsh.at[pl.ds(s * rps, rps)])
        pltpu.sync_copy(z_ref, dsh.at[pl.ds(s * rps, rps)])
        plsc.subcore_barrier()
        base0 = wid * per_w

        def start_load(i, slot):
            loc = base0 + i * ch
            pltpu.async_copy(dst_ref.at[pl.ds(base + loc, ch)],
                             idx_v.at[slot], sem_l)
            pltpu.async_copy(wm_ref.at[pl.ds(loc, ch)], wm_v.at[slot], sem_l)
            pltpu.async_copy(wx_ref.at[pl.ds(loc, ch)], wx_v.at[slot], sem_l)

        def wait_load():
            pltpu.make_async_copy(dst_ref.at[pl.ds(0, ch)], idx_v.at[0],
                                  sem_l).wait()
            pltpu.make_async_copy(wm_ref.at[pl.ds(0, ch)], wm_v.at[0],
                                  sem_l).wait()
            pltpu.make_async_copy(wx_ref.at[pl.ds(0, ch)], wx_v.at[0],
                                  sem_l).wait()

        def start_scat(slot):
            pltpu.async_copy(wm_v.at[slot], hsh.at[idx_v.at[slot]], sem_s,
                             add=True)
            pltpu.async_copy(wx_v.at[slot], dsh.at[idx_v.at[slot]], sem_s,
                             add=True)

        def wait_scat():
            pltpu.make_async_copy(wm_v.at[0], hsh.at[idx_v.at[0]],
                                  sem_s).wait()
            pltpu.make_async_copy(wx_v.at[0], dsh.at[idx_v.at[0]],
                                  sem_s).wait()

        def half(i, slot):
            wait_load()
            start_scat(slot)

            @pl.when(jnp.logical_and(i >= 1, i + 1 < n_ch))
            def _():
                wait_scat()

            @pl.when(i + 1 < n_ch)
            def _():
                start_load(i + 1, 1 - slot)

        def it2(j, carry):
            half(2 * j, 0)
            half(2 * j + 1, 1)
            return carry

        start_load(0, 0)
        jax.lax.fori_loop(0, n_ch // 2, it2, 0)
        wait_scat()
        wait_scat()
        plsc.subcore_barrier()
        pltpu.sync_copy(hsh.at[pl.ds(s * rps, rps)],
                        hs_out.at[c, pl.ds(s * rps, rps)])
        pltpu.sync_copy(dsh.at[pl.ds(s * rps, rps)],
                        ds_out.at[c, pl.ds(s * rps, rps)])

    hsP, dsP = pl.kernel(
        body, out_type=out_type, mesh=_sc_mesh(), scratch_types=scratch,
        compiler_params=pltpu.CompilerParams(use_tc_tiling_on_sc=False),
    )(dst, wmsg, wex, zeros)
    return hsP[0], hsP[1], dsP[0], dsP[1]


def kernel(node_features, edge_features, params, edge_index, agent_index):
    p = params
    f32 = jnp.float32
    src = edge_index[0]
    dst = edge_index[1]

    M = _kron8(jnp.full((HID, HID), 1.0 / HID, f32))
    SX = _kron8(jnp.kron(jnp.eye(4, dtype=f32), jnp.ones((4, 4), f32)))

    WinA = jnp.zeros((HID, HID), f32).at[:NODE_IN].set(p['W_in'][:NODE_IN])
    WinB = jnp.zeros((HID, HID), f32).at[:AGENT_DIM].set(p['W_in'][NODE_IN:])
    nfp = jnp.pad(node_features, ((0, 0), (0, HID - NODE_IN)))
    afp = jnp.pad(p['agent_features'], ((0, 0), (0, HID - AGENT_DIM)))
    afg = _gather_agent(afp, agent_index)
    bt = _t8(p['b_in'] + p['b_pe'])

    h8, q08, k08, v08 = _tc_full(_prep_body, 4)(
        nfp.reshape(RN, 128), afg.reshape(RN, 128),
        p['positional_embedding'].reshape(RN, 128),
        _kron8(WinA), _kron8(WinB), _kron8(p['W_pe']), bt,
        _kron8(p['Wq'][0] * 0.5), _t8(p['bq'][0] * 0.5),
        _kron8(p['Wk'][0]), _t8(p['bk'][0]),
        _kron8(p['Wv'][0]), _t8(p['bv'][0]))

    NS = 1
    ES = E // NS
    RS = RE // NS
    qt0 = q08.reshape(N, HID)
    kt0 = k08.reshape(N, HID)
    vt0 = v08.reshape(N, HID)

    R = jnp.kron(jnp.eye(8, dtype=f32),
                 p['W_e_in'].astype(f32)).astype(jnp.bfloat16)
    c0_w = (R, _t8(p['b_e_in']),
            _kron8(p['We'][0]), _t8(p['be'][0]), SX,
            _kron8(p['Woe'][0]), _t8(p['boe'][0]),
            _kron8(p['Wef1'][0]), _t8(p['bef1'][0]),
            _kron8(p['Wef2'][0]), _t8(p['bef2'][0]),
            _t8(p['lne1_s'][0]), _t8(p['lne1_b'][0]),
            _t8(p['lne2_s'][0]), _t8(p['lne2_b'][0]), M)
    ef8 = edge_features.reshape(RE, 8)
    zeros = jnp.zeros((N // 16, HID), f32)

    g0 = [_sc_gather([qt0, kt0, vt0], [dst, src], [0, 1, 1], ES, 1000,
                     base=s * ES, fold01=True) for s in range(NS)]
    c0 = [_tc_edge(_c0_body, [8, 128, 128], [128, 128, 128], RS,
                   out_dtypes=[jnp.bfloat16, jnp.float32, jnp.float32])(
        ef8[s * RS:(s + 1) * RS], g0[s][0].reshape(RS, 128),
        g0[s][1].reshape(RS, 128), *c0_w)
        for s in range(NS)]
    sc = [_sc_scatter(dst, c0[s][1].reshape(ES, HID),
                      c0[s][2].reshape(ES, HID), zeros, ES, base=s * ES)
          for s in range(NS)]

    zRN = jnp.zeros((RN, 128), f32)
    hs_args = [sc[s][i].reshape(RN, 128) for s in range(NS) for i in (0, 1)]
    ds_args = [sc[s][i].reshape(RN, 128) for s in range(NS) for i in (2, 3)]
    hs_args += [zRN] * (4 - len(hs_args))
    ds_args += [zRN] * (4 - len(ds_args))
    q18, k18 = _tc_full(_n0_body, 2)(
        h8, *hs_args, *ds_args,
        _kron8(p['Wo'][0]), _t8(p['bo'][0]),
        _kron8(p['Wf1'][0]), _t8(p['bf1'][0]),
        _kron8(p['Wf2'][0]), _t8(p['bf2'][0]),
        _t8(p['ln1_s'][0]), _t8(p['ln1_b'][0]),
        _t8(p['ln2_s'][0]), _t8(p['ln2_b'][0]), M,
        _kron8(p['Wq'][1] * 0.5), _t8(p['bq'][1] * 0.5),
        _kron8(p['Wk'][1]), _t8(p['bk'][1]))

    Wout8 = jnp.kron(jnp.eye(8, dtype=f32),
                     p['W_out'].astype(f32)).astype(jnp.bfloat16)
    bout = jnp.tile(p['b_out'].astype(f32), 8).reshape(1, 8)
    c1_w = (_kron8(p['We'][1]), _t8(p['be'][1]),
            _kron8(p['Woe'][1]), _t8(p['boe'][1]),
            _kron8(p['Wef1'][1]), _t8(p['bef1'][1]),
            _kron8(p['Wef2'][1]), _t8(p['bef2'][1]),
            _t8(p['lne1_s'][1]), _t8(p['lne1_b'][1]),
            _t8(p['lne2_s'][1]), _t8(p['lne2_b'][1]), M,
            Wout8, bout)
    qt1 = q18.reshape(N, HID)
    kt1 = k18.reshape(N, HID)

    g1 = [_sc_gather([qt1, kt1], [dst, src], [0, 1], ES, 1000, base=s * ES,
                     fold01=True)
          for s in range(NS)]
    esc = [_tc_edge(_c1_body, [128, 128], [8], RS)(
        c0[s][0], g1[s][0].reshape(RS, 128), *c1_w)[0]
        for s in range(NS)]

    return jnp.concatenate(esc, axis=0).reshape(E)


# speedup vs baseline: 1.3759x; 1.0501x over previous
import functools

import jax
import jax.numpy as jnp
import numpy as np
from jax.experimental import pallas as pl
from jax.experimental.pallas import tpu as pltpu
from jax.experimental.pallas import tpu_sc as plsc

N = 10000
E = 640000
HID = 16
NODE_IN = 10
AGENT_DIM = 5
RE = E // 8
RN = N // 8
BB = 3200
GRID_E = RE // BB

_INTERPRET = False


def _kron8(w):
    return jnp.kron(jnp.eye(8, dtype=jnp.float32),
                    w.astype(jnp.float32)).astype(jnp.bfloat16)


def _t8(b):
    return jnp.tile(b.astype(jnp.float32), 8).reshape(1, -1)


def _wspec(a):
    n = a.ndim
    return pl.BlockSpec(a.shape, lambda i, _n=n: (0,) * _n)


def _espec(minor):
    return pl.BlockSpec((BB, minor), lambda i: (i, 0))


def _dot(a, b):
    return jnp.dot(a.astype(jnp.bfloat16), b,
                   preferred_element_type=jnp.float32)


def _ln(x, M, s, b):
    mu = _dot(x, M)
    xc = x - mu
    var = _dot(xc * xc, M)
    return xc * jax.lax.rsqrt(var + 1e-5) * s + b


def _prep_body(nfp, afg, pe, WA, WB, WP, bt, Wq, bqt, Wk, bkt, Wv, bvt,
               h_o, q_o, k_o, v_o):
    h = (_dot(nfp[...], WA[...]) + _dot(afg[...], WB[...])
         + _dot(pe[...], WP[...]) + bt[...])
    h_o[...] = h
    q_o[...] = _dot(h, Wq[...]) + bqt[...]
    k_o[...] = _dot(h, Wk[...]) + bkt[...]
    v_o[...] = _dot(h, Wv[...]) + bvt[...]


def _c0_body(ef8, qk, vs, R, bet, We, bewt, SX, Woe, boet,
             W1, b1t, W2, b2t, s1, bb1, s2, bb2, M,
             e1_o, wm_o, wx_o):
    e = _dot(ef8[...], R[...]) + bet[...]
    ew = _dot(e, We[...]) + bewt[...]
    score = qk[...] * ew
    wex = jnp.exp(jnp.clip(_dot(score, SX[...]), -5.0, 5.0))
    gate = jax.nn.sigmoid(ew)
    wm_o[...] = wex * vs[...] * gate
    wx_o[...] = wex
    x = _ln(e + _dot(score, Woe[...]) + boet[...], M[...], s1[...], bb1[...])
    f = _dot(jax.nn.relu(_dot(x, W1[...]) + b1t[...]), W2[...]) + b2t[...]
    e1_o[...] = _ln(x + f, M[...], s2[...], bb2[...]).astype(jnp.bfloat16)


def _n0_body(h, hs0, hs1, hs2, hs3, ds0, ds1, ds2, ds3, Wo, bot,
             W1, b1t, W2, b2t, s1, bb1, s2, bb2, M, Wq, bqt, Wk, bkt,
             q_o, k_o):
    hagg = ((hs0[...] + hs1[...]) + (hs2[...] + hs3[...])) / (
        (ds0[...] + ds1[...]) + (ds2[...] + ds3[...]) + 1e-9)
    hn = _dot(hagg, Wo[...]) + bot[...]
    x = _ln(h[...] + hn, M[...], s1[...], bb1[...])
    f = _dot(jax.nn.relu(_dot(x, W1[...]) + b1t[...]), W2[...]) + b2t[...]
    x = _ln(x + f, M[...], s2[...], bb2[...])
    q_o[...] = _dot(x, Wq[...]) + bqt[...]
    k_o[...] = _dot(x, Wk[...]) + bkt[...]


def _c1_body(e1, qk, We, bewt, Woe, boet, W1, b1t, W2, b2t,
             s1, bb1, s2, bb2, M, Wout, bout,
             esc_o):
    e = e1[...].astype(jnp.float32)
    ew = _dot(e, We[...]) + bewt[...]
    score = qk[...] * ew
    x = _ln(e + _dot(score, Woe[...]) + boet[...], M[...], s1[...], bb1[...])
    f = _dot(jax.nn.relu(_dot(x, W1[...]) + b1t[...]), W2[...]) + b2t[...]
    e2 = _ln(x + f, M[...], s2[...], bb2[...])
    esc_o[...] = jax.nn.sigmoid(_dot(e2, Wout[...]) + bout[...])


def _tc_full(body, n_out):
    def run(*args):
        out_shape = tuple(jax.ShapeDtypeStruct((RN, 128), jnp.float32)
                          for _ in range(n_out))
        return pl.pallas_call(body, out_shape=out_shape,
                              interpret=_INTERPRET)(*args)
    return run


def _tc_edge(body, in_minors, out_minors, rows, out_dtypes=None):
    def run(*args):
        n_data = len(in_minors)
        in_specs = [_espec(m) for m in in_minors]
        in_specs += [_wspec(a) for a in args[n_data:]]
        out_specs = tuple(_espec(m) for m in out_minors)
        dts = out_dtypes or [jnp.float32] * len(out_minors)
        out_shape = tuple(jax.ShapeDtypeStruct((rows, m), dt)
                          for m, dt in zip(out_minors, dts))
        return pl.pallas_call(body, grid=(rows // BB,), in_specs=in_specs,
                              out_specs=out_specs, out_shape=out_shape,
                              interpret=_INTERPRET)(*args)
    return run


_NW = 32


def _sc_mesh():
    return plsc.VectorSubcoreMesh(core_axis_name="c", subcore_axis_name="s")


def _sc_gather(tables, idxs, tmap, total, ch, base=0, fold01=False):
    n_t = len(tables)
    n_u = len(idxs)
    per_w = total // _NW
    n_ch = per_w // ch
    n_o = n_t - 1 if fold01 else n_t
    omap = ([0] + list(range(2, n_t))) if fold01 else list(range(n_t))
    out_type = tuple(jax.ShapeDtypeStruct((total, HID), jnp.float32)
                     for _ in range(n_o))
    scratch = ([pltpu.VMEM((2, ch), jnp.int32) for _ in range(n_u)]
               + [pltpu.VMEM((2, ch, HID), jnp.float32) for _ in range(n_t)]
               + [pltpu.SemaphoreType.DMA] * 3)

    def body(*refs):
        t_refs = refs[:n_t]
        i_refs = refs[n_t:n_t + n_u]
        o_refs = refs[n_t + n_u:n_t + n_u + n_o]
        iv = refs[n_t + n_u + n_o:n_t + 2 * n_u + n_o]
        rv = refs[n_t + 2 * n_u + n_o:2 * n_t + 2 * n_u + n_o]
        sem_i, sem_g, sem_w = refs[-3:]
        wid = jax.lax.axis_index("s") * 2 + jax.lax.axis_index("c")
        base0 = wid * per_w

        def start_idx(i, slot):
            for u in range(n_u):
                pltpu.async_copy(i_refs[u].at[pl.ds(base + base0 + i * ch,
                                                    ch)],
                                 iv[u].at[slot], sem_i)

        def wait_idx():
            for u in range(n_u):
                pltpu.make_async_copy(i_refs[u].at[pl.ds(0, ch)],
                                      iv[u].at[0], sem_i).wait()

        def start_gather(i, slot):
            for t in range(n_t):
                pltpu.async_copy(t_refs[t].at[iv[tmap[t]].at[slot]],
                                 rv[t].at[slot], sem_g)

        def wait_gather():
            for t in range(n_t):
                pltpu.make_async_copy(t_refs[t].at[iv[tmap[t]].at[0]],
                                      rv[t].at[0], sem_g).wait()

        def mul01(slot):
            if not fold01:
                return

            def mbody(r, c):
                for u2 in range(4):
                    i2 = r * 4 + u2
                    rv[0][slot, i2, :] = rv[0][slot, i2, :] * rv[1][slot, i2, :]
                return c

            jax.lax.fori_loop(0, ch // 4, mbody, 0)

        def start_wb(i, slot):
            for j in range(n_o):
                pltpu.async_copy(rv[omap[j]].at[slot],
                                 o_refs[j].at[pl.ds(base0 + i * ch, ch)],
                                 sem_w)

        def wait_wb():
            for j in range(n_o):
                pltpu.make_async_copy(rv[omap[j]].at[0],
                                      o_refs[j].at[pl.ds(0, ch)], sem_w).wait()

        if n_ch < 2:
            for i in range(n_ch):
                start_idx(i, 0)
                wait_idx()
                start_gather(i, 0)
                wait_gather()
                mul01(0)
                start_wb(i, 0)
                wait_wb()
            return

        def half(i, slot):
            @pl.when(i > 0)
            def _():
                wait_gather()

            wait_idx()

            @pl.when(i >= 2)
            def _():
                wait_wb()

            start_gather(i, slot)

            @pl.when(i > 0)
            def _():
                mul01(1 - slot)
                start_wb(i - 1, 1 - slot)

            @pl.when(i + 1 < n_ch)
            def _():
                start_idx(i + 1, 1 - slot)

        def it2(j, carry):
            half(2 * j, 0)
            half(2 * j + 1, 1)
            return carry

        start_idx(0, 0)
        jax.lax.fori_loop(0, n_ch // 2, it2, 0)
        wait_gather()
        mul01((n_ch - 1) % 2)
        start_wb(n_ch - 1, (n_ch - 1) % 2)
        wait_wb()
        wait_wb()

    return pl.kernel(
        body, out_type=out_type, mesh=_sc_mesh(), scratch_types=scratch,
        compiler_params=pltpu.CompilerParams(use_tc_tiling_on_sc=False),
    )(*tables, *idxs)


def _gather_agent(afp, agent_index):
    npad = 10240
    idx = jnp.pad(agent_index, (0, npad - N))
    (out,) = _sc_gather([afp], [idx], [0], npad, 320)
    return out[:N]


def _sc_scatter(dst, wmsg, wex, zeros, total, base=0):
    ch = 1000
    per_w = total // _NW
    n_ch = per_w // ch
    rps = N // 16
    out_type = (jax.ShapeDtypeStruct((2, N, HID), jnp.float32),
                jax.ShapeDtypeStruct((2, N, HID), jnp.float32))
    scratch = [pltpu.VMEM((2, ch), jnp.int32),
               pltpu.VMEM((2, ch, HID), jnp.float32),
               pltpu.VMEM((2, ch, HID), jnp.float32),
               pltpu.VMEM_SHARED((N, HID), jnp.float32),
               pltpu.VMEM_SHARED((N, HID), jnp.float32),
               pltpu.SemaphoreType.DMA,
               pltpu.SemaphoreType.DMA]

    def body(dst_ref, wm_ref, wx_ref, z_ref, hs_out, ds_out,
             idx_v, wm_v, wx_v, hsh, dsh, sem_l, sem_s):
        c = jax.lax.axis_index("c")
        s = jax.lax.axis_index("s")
        wid = s * 2 + c
        pltpu.sync_copy(z_ref, hsh.at[pl.ds(s * rps, rps)])
        pltpu.sync_copy(z_ref, dsh.at[pl.ds(s * rps, rps)])
        plsc.subcore_barrier()
        base0 = wid * per_w

        def start_load(i, slot):
            loc = base0 + i * ch
            pltpu.async_copy(dst_ref.at[pl.ds(base + loc, ch)],
                             idx_v.at[slot], sem_l)
            pltpu.async_copy(wm_ref.at[pl.ds(loc, ch)], wm_v.at[slot], sem_l)
            pltpu.async_copy(wx_ref.at[pl.ds(loc, ch)], wx_v.at[slot], sem_l)

        def wait_load():
            pltpu.make_async_copy(dst_ref.at[pl.ds(0, ch)], idx_v.at[0],
                                  sem_l).wait()
            pltpu.make_async_copy(wm_ref.at[pl.ds(0, ch)], wm_v.at[0],
                                  sem_l).wait()
            pltpu.make_async_copy(wx_ref.at[pl.ds(0, ch)], wx_v.at[0],
                                  sem_l).wait()

        def start_scat(slot):
            pltpu.async_copy(wm_v.at[slot], hsh.at[idx_v.at[slot]], sem_s,
                             add=True)
            pltpu.async_copy(wx_v.at[slot], dsh.at[idx_v.at[slot]], sem_s,
                             add=True)

        def wait_scat():
            pltpu.make_async_copy(wm_v.at[0], hsh.at[idx_v.at[0]],
                                  sem_s).wait()
            pltpu.make_async_copy(wx_v.at[0], dsh.at[idx_v.at[0]],
                                  sem_s).wait()

        def half(i, slot):
            wait_load()
            start_scat(slot)

            @pl.when(jnp.logical_and(i >= 1, i + 1 < n_ch))
            def _():
                wait_scat()

            @pl.when(i + 1 < n_ch)
            def _():
                start_load(i + 1, 1 - slot)

        def it2(j, carry):
            half(2 * j, 0)
            half(2 * j + 1, 1)
            return carry

        start_load(0, 0)
        jax.lax.fori_loop(0, n_ch // 2, it2, 0)
        wait_scat()
        wait_scat()
        plsc.subcore_barrier()
        pltpu.sync_copy(hsh.at[pl.ds(s * rps, rps)],
                        hs_out.at[c, pl.ds(s * rps, rps)])
        pltpu.sync_copy(dsh.at[pl.ds(s * rps, rps)],
                        ds_out.at[c, pl.ds(s * rps, rps)])

    hsP, dsP = pl.kernel(
        body, out_type=out_type, mesh=_sc_mesh(), scratch_types=scratch,
        compiler_params=pltpu.CompilerParams(use_tc_tiling_on_sc=False),
    )(dst, wmsg, wex, zeros)
    return hsP[0], hsP[1], dsP[0], dsP[1]


def kernel(node_features, edge_features, params, edge_index, agent_index):
    p = params
    f32 = jnp.float32
    src = edge_index[0]
    dst = edge_index[1]

    M = _kron8(jnp.full((HID, HID), 1.0 / HID, f32))
    SX = _kron8(jnp.kron(jnp.eye(4, dtype=f32), jnp.ones((4, 4), f32)))

    WinA = jnp.zeros((HID, HID), f32).at[:NODE_IN].set(p['W_in'][:NODE_IN])
    WinB = jnp.zeros((HID, HID), f32).at[:AGENT_DIM].set(p['W_in'][NODE_IN:])
    nfp = jnp.pad(node_features, ((0, 0), (0, HID - NODE_IN)))
    afp = jnp.pad(p['agent_features'], ((0, 0), (0, HID - AGENT_DIM)))
    afg = _gather_agent(afp, agent_index)
    bt = _t8(p['b_in'] + p['b_pe'])

    h8, q08, k08, v08 = _tc_full(_prep_body, 4)(
        nfp.reshape(RN, 128), afg.reshape(RN, 128),
        p['positional_embedding'].reshape(RN, 128),
        _kron8(WinA), _kron8(WinB), _kron8(p['W_pe']), bt,
        _kron8(p['Wq'][0] * 0.5), _t8(p['bq'][0] * 0.5),
        _kron8(p['Wk'][0]), _t8(p['bk'][0]),
        _kron8(p['Wv'][0]), _t8(p['bv'][0]))

    NS = 1
    ES = E // NS
    RS = RE // NS
    qt0 = q08.reshape(N, HID)
    kt0 = k08.reshape(N, HID)
    vt0 = v08.reshape(N, HID)

    R = jnp.kron(jnp.eye(8, dtype=f32),
                 p['W_e_in'].astype(f32)).astype(jnp.bfloat16)
    c0_w = (R, _t8(p['b_e_in']),
            _kron8(p['We'][0]), _t8(p['be'][0]), SX,
            _kron8(p['Woe'][0]), _t8(p['boe'][0]),
            _kron8(p['Wef1'][0]), _t8(p['bef1'][0]),
            _kron8(p['Wef2'][0]), _t8(p['bef2'][0]),
            _t8(p['lne1_s'][0]), _t8(p['lne1_b'][0]),
            _t8(p['lne2_s'][0]), _t8(p['lne2_b'][0]), M)
    ef8 = edge_features.reshape(RE, 8)
    zeros = jnp.zeros((N // 16, HID), f32)

    g0 = [_sc_gather([qt0, kt0, vt0], [dst, src], [0, 1, 1], ES, 1000,
                     base=s * ES, fold01=True) for s in range(NS)]
    c0 = [_tc_edge(_c0_body, [8, 128, 128], [128, 128, 128], RS,
                   out_dtypes=[jnp.bfloat16, jnp.float32, jnp.float32])(
        ef8[s * RS:(s + 1) * RS], g0[s][0].reshape(RS, 128),
        g0[s][1].reshape(RS, 128), *c0_w)
        for s in range(NS)]
    sc = [_sc_scatter(dst, c0[s][1].reshape(ES, HID),
                      c0[s][2].reshape(ES, HID), zeros, ES, base=s * ES)
          for s in range(NS)]

    zRN = jnp.zeros((RN, 128), f32)
    hs_args = [sc[s][i].reshape(RN, 128) for s in range(NS) for i in (0, 1)]
    ds_args = [sc[s][i].reshape(RN, 128) for s in range(NS) for i in (2, 3)]
    hs_args += [zRN] * (4 - len(hs_args))
    ds_args += [zRN] * (4 - len(ds_args))
    q18, k18 = _tc_full(_n0_body, 2)(
        h8, *hs_args, *ds_args,
        _kron8(p['Wo'][0]), _t8(p['bo'][0]),
        _kron8(p['Wf1'][0]), _t8(p['bf1'][0]),
        _kron8(p['Wf2'][0]), _t8(p['bf2'][0]),
        _t8(p['ln1_s'][0]), _t8(p['ln1_b'][0]),
        _t8(p['ln2_s'][0]), _t8(p['ln2_b'][0]), M,
        _kron8(p['Wq'][1] * 0.5), _t8(p['bq'][1] * 0.5),
        _kron8(p['Wk'][1]), _t8(p['bk'][1]))

    Wout8 = jnp.kron(jnp.eye(8, dtype=f32),
                     p['W_out'].astype(f32)).astype(jnp.bfloat16)
    bout = jnp.tile(p['b_out'].astype(f32), 8).reshape(1, 8)
    c1_w = (_kron8(p['We'][1]), _t8(p['be'][1]),
            _kron8(p['Woe'][1]), _t8(p['boe'][1]),
            _kron8(p['Wef1'][1]), _t8(p['bef1'][1]),
            _kron8(p['Wef2'][1]), _t8(p['bef2'][1]),
            _t8(p['lne1_s'][1]), _t8(p['lne1_b'][1]),
            _t8(p['lne2_s'][1]), _t8(p['lne2_b'][1]), M,
            Wout8, bout)
    qt1 = q18.reshape(N, HID)
    kt1 = k18.reshape(N, HID)

    g1 = [_sc_gather([qt1, kt1], [dst, src], [0, 1], ES, 1000, base=s * ES,
                     fold01=True)
          for s in range(NS)]
    esc = [_tc_edge(_c1_body, [128, 128], [8], RS)(
        c0[s][0], g1[s][0].reshape(RS, 128), *c1_w)[0]
        for s in range(NS)]

    return jnp.concatenate(esc, axis=0).reshape(E)


# speedup vs baseline: 1.3759x; 1.0000x over previous
import jax
import jax.numpy as jnp
from jax.experimental import pallas as pl
from jax.experimental.pallas import tpu as pltpu
from jax.experimental.pallas import tpu_sc as plsc

N = 10000
E = 640000
HID = 16
NODE_IN = 10
AGENT_DIM = 5
RE = E // 8
RN = N // 8
BB = 3200
GRID_E = RE // BB


def _kron8(w):
    return jnp.kron(jnp.eye(8, dtype=jnp.float32),
                    w.astype(jnp.float32)).astype(jnp.bfloat16)


def _t8(b):
    return jnp.tile(b.astype(jnp.float32), 8).reshape(1, -1)


def _wspec(a):
    n = a.ndim
    return pl.BlockSpec(a.shape, lambda i, _n=n: (0,) * _n)


def _espec(minor):
    return pl.BlockSpec((BB, minor), lambda i: (i, 0))


def _dot(a, b):
    return jnp.dot(a.astype(jnp.bfloat16), b,
                   preferred_element_type=jnp.float32)


def _ln(x, M, s, b):
    mu = _dot(x, M)
    xc = x - mu
    var = _dot(xc * xc, M)
    return xc * jax.lax.rsqrt(var + 1e-5) * s + b


def _prep_body(nfp, afg, pe, WA, WB, WP, bt, Wq, bqt, Wk, bkt, Wv, bvt,
               h_o, q_o, k_o, v_o):
    h = (_dot(nfp[...], WA[...]) + _dot(afg[...], WB[...])
         + _dot(pe[...], WP[...]) + bt[...])
    h_o[...] = h
    q_o[...] = _dot(h, Wq[...]) + bqt[...]
    k_o[...] = _dot(h, Wk[...]) + bkt[...]
    v_o[...] = _dot(h, Wv[...]) + bvt[...]


def _c0_body(ef8, qk, vs, R, bet, We, bewt, SX, Woe, boet,
             W1, b1t, W2, b2t, s1, bb1, s2, bb2, M,
             e1_o, wm_o, wx_o):
    e = _dot(ef8[...], R[...]) + bet[...]
    ew = _dot(e, We[...]) + bewt[...]
    score = qk[...] * ew
    wex = jnp.exp(jnp.clip(_dot(score, SX[...]), -5.0, 5.0))
    gate = jax.nn.sigmoid(ew)
    wm_o[...] = wex * vs[...] * gate
    wx_o[...] = wex
    x = _ln(e + _dot(score, Woe[...]) + boet[...], M[...], s1[...], bb1[...])
    f = _dot(jax.nn.relu(_dot(x, W1[...]) + b1t[...]), W2[...]) + b2t[...]
    e1_o[...] = _ln(x + f, M[...], s2[...], bb2[...]).astype(jnp.bfloat16)


def _n0_body(h, hs0, hs1, hs2, hs3, ds0, ds1, ds2, ds3, Wo, bot,
             W1, b1t, W2, b2t, s1, bb1, s2, bb2, M, Wq, bqt, Wk, bkt,
             q_o, k_o):
    hagg = ((hs0[...] + hs1[...]) + (hs2[...] + hs3[...])) / (
        (ds0[...] + ds1[...]) + (ds2[...] + ds3[...]) + 1e-9)
    hn = _dot(hagg, Wo[...]) + bot[...]
    x = _ln(h[...] + hn, M[...], s1[...], bb1[...])
    f = _dot(jax.nn.relu(_dot(x, W1[...]) + b1t[...]), W2[...]) + b2t[...]
    x = _ln(x + f, M[...], s2[...], bb2[...])
    q_o[...] = _dot(x, Wq[...]) + bqt[...]
    k_o[...] = _dot(x, Wk[...]) + bkt[...]


def _c1_body(e1, qk, We, bewt, Woe, boet, W1, b1t, W2, b2t,
             s1, bb1, s2, bb2, M, Wout, bout,
             esc_o):
    e = e1[...].astype(jnp.float32)
    ew = _dot(e, We[...]) + bewt[...]
    score = qk[...] * ew
    x = _ln(e + _dot(score, Woe[...]) + boet[...], M[...], s1[...], bb1[...])
    f = _dot(jax.nn.relu(_dot(x, W1[...]) + b1t[...]), W2[...]) + b2t[...]
    e2 = _ln(x + f, M[...], s2[...], bb2[...])
    esc_o[...] = jax.nn.sigmoid(_dot(e2, Wout[...]) + bout[...])


def _tc_full(body, n_out):
    def run(*args):
        out_shape = tuple(jax.ShapeDtypeStruct((RN, 128), jnp.float32)
                          for _ in range(n_out))
        return pl.pallas_call(body, out_shape=out_shape)(*args)
    return run


def _tc_edge(body, in_minors, out_minors, rows, out_dtypes=None):
    def run(*args):
        n_data = len(in_minors)
        in_specs = [_espec(m) for m in in_minors]
        in_specs += [_wspec(a) for a in args[n_data:]]
        out_specs = tuple(_espec(m) for m in out_minors)
        dts = out_dtypes or [jnp.float32] * len(out_minors)
        out_shape = tuple(jax.ShapeDtypeStruct((rows, m), dt)
                          for m, dt in zip(out_minors, dts))
        return pl.pallas_call(body, grid=(rows // BB,), in_specs=in_specs,
                              out_specs=out_specs, out_shape=out_shape)(*args)
    return run


_NW = 32


def _sc_mesh():
    return plsc.VectorSubcoreMesh(core_axis_name="c", subcore_axis_name="s")


def _sc_gather(tables, idxs, tmap, total, ch, base=0, fold01=False):
    n_t = len(tables)
    n_u = len(idxs)
    per_w = total // _NW
    n_ch = per_w // ch
    n_o = n_t - 1 if fold01 else n_t
    omap = ([0] + list(range(2, n_t))) if fold01 else list(range(n_t))
    out_type = tuple(jax.ShapeDtypeStruct((total, HID), jnp.float32)
                     for _ in range(n_o))
    scratch = ([pltpu.VMEM((2, ch), jnp.int32) for _ in range(n_u)]
               + [pltpu.VMEM((2, ch, HID), jnp.float32) for _ in range(n_t)]
               + [pltpu.SemaphoreType.DMA] * 3)

    def body(*refs):
        t_refs = refs[:n_t]
        i_refs = refs[n_t:n_t + n_u]
        o_refs = refs[n_t + n_u:n_t + n_u + n_o]
        iv = refs[n_t + n_u + n_o:n_t + 2 * n_u + n_o]
        rv = refs[n_t + 2 * n_u + n_o:2 * n_t + 2 * n_u + n_o]
        sem_i, sem_g, sem_w = refs[-3:]
        wid = jax.lax.axis_index("s") * 2 + jax.lax.axis_index("c")
        base0 = wid * per_w

        def start_idx(i, slot):
            for u in range(n_u):
                pltpu.async_copy(i_refs[u].at[pl.ds(base + base0 + i * ch,
                                                    ch)],
                                 iv[u].at[slot], sem_i)

        def wait_idx():
            for u in range(n_u):
                pltpu.make_async_copy(i_refs[u].at[pl.ds(0, ch)],
                                      iv[u].at[0], sem_i).wait()

        def start_gather(i, slot):
            for t in range(n_t):
                pltpu.async_copy(t_refs[t].at[iv[tmap[t]].at[slot]],
                                 rv[t].at[slot], sem_g)

        def wait_gather():
            for t in range(n_t):
                pltpu.make_async_copy(t_refs[t].at[iv[tmap[t]].at[0]],
                                      rv[t].at[0], sem_g).wait()

        def mul01(slot):
            if not fold01:
                return

            def mbody(r, c):
                for u2 in range(4):
                    i2 = r * 4 + u2
                    rv[0][slot, i2, :] = rv[0][slot, i2, :] * rv[1][slot, i2, :]
                return c

            jax.lax.fori_loop(0, ch // 4, mbody, 0)

        def start_wb(i, slot):
            for j in range(n_o):
                pltpu.async_copy(rv[omap[j]].at[slot],
                                 o_refs[j].at[pl.ds(base0 + i * ch, ch)],
                                 sem_w)

        def wait_wb():
            for j in range(n_o):
                pltpu.make_async_copy(rv[omap[j]].at[0],
                                      o_refs[j].at[pl.ds(0, ch)], sem_w).wait()

        if n_ch < 2:
            for i in range(n_ch):
                start_idx(i, 0)
                wait_idx()
                start_gather(i, 0)
                wait_gather()
                mul01(0)
                start_wb(i, 0)
                wait_wb()
            return

        def half(i, slot):
            @pl.when(i > 0)
            def _():
                wait_gather()

            wait_idx()

            @pl.when(i >= 2)
            def _():
                wait_wb()

            start_gather(i, slot)

            @pl.when(i > 0)
            def _():
                mul01(1 - slot)
                start_wb(i - 1, 1 - slot)

            @pl.when(i + 1 < n_ch)
            def _():
                start_idx(i + 1, 1 - slot)

        def it2(j, carry):
            half(2 * j, 0)
            half(2 * j + 1, 1)
            return carry

        start_idx(0, 0)
        jax.lax.fori_loop(0, n_ch // 2, it2, 0)
        wait_gather()
        mul01((n_ch - 1) % 2)
        start_wb(n_ch - 1, (n_ch - 1) % 2)
        wait_wb()
        wait_wb()

    return pl.kernel(
        body, out_type=out_type, mesh=_sc_mesh(), scratch_types=scratch,
        compiler_params=pltpu.CompilerParams(use_tc_tiling_on_sc=False),
    )(*tables, *idxs)


def _gather_agent(afp, agent_index):
    npad = 10240
    idx = jnp.pad(agent_index, (0, npad - N))
    (out,) = _sc_gather([afp], [idx], [0], npad, 320)
    return out[:N]


def _sc_scatter(dst, wmsg, wex, zeros, total, base=0):
    ch = 1000
    per_w = total // _NW
    n_ch = per_w // ch
    rps = N // 16
    out_type = (jax.ShapeDtypeStruct((2, N, HID), jnp.float32),
                jax.ShapeDtypeStruct((2, N, HID), jnp.float32))
    scratch = [pltpu.VMEM((2, ch), jnp.int32),
               pltpu.VMEM((2, ch, HID), jnp.float32),
               pltpu.VMEM((2, ch, HID), jnp.float32),
               pltpu.VMEM_SHARED((N, HID), jnp.float32),
               pltpu.VMEM_SHARED((N, HID), jnp.float32),
               pltpu.SemaphoreType.DMA,
               pltpu.SemaphoreType.DMA]

    def body(dst_ref, wm_ref, wx_ref, z_ref, hs_out, ds_out,
             idx_v, wm_v, wx_v, hsh, dsh, sem_l, sem_s):
        c = jax.lax.axis_index("c")
        s = jax.lax.axis_index("s")
        wid = s * 2 + c
        pltpu.sync_copy(z_ref, hsh.at[pl.ds(s * rps, rps)])
        pltpu.sync_copy(z_ref, dsh.at[pl.ds(s * rps, rps)])
        plsc.subcore_barrier()
        base0 = wid * per_w

        def start_load(i, slot):
            loc = base0 + i * ch
            pltpu.async_copy(dst_ref.at[pl.ds(base + loc, ch)],
                             idx_v.at[slot], sem_l)
            pltpu.async_copy(wm_ref.at[pl.ds(loc, ch)], wm_v.at[slot], sem_l)
            pltpu.async_copy(wx_ref.at[pl.ds(loc, ch)], wx_v.at[slot], sem_l)

        def wait_load():
            pltpu.make_async_copy(dst_ref.at[pl.ds(0, ch)], idx_v.at[0],
                                  sem_l).wait()
            pltpu.make_async_copy(wm_ref.at[pl.ds(0, ch)], wm_v.at[0],
                                  sem_l).wait()
            pltpu.make_async_copy(wx_ref.at[pl.ds(0, ch)], wx_v.at[0],
                                  sem_l).wait()

        def start_scat(slot):
            pltpu.async_copy(wm_v.at[slot], hsh.at[idx_v.at[slot]], sem_s,
                             add=True)
            pltpu.async_copy(wx_v.at[slot], dsh.at[idx_v.at[slot]], sem_s,
                             add=True)

        def wait_scat():
            pltpu.make_async_copy(wm_v.at[0], hsh.at[idx_v.at[0]],
                                  sem_s).wait()
            pltpu.make_async_copy(wx_v.at[0], dsh.at[idx_v.at[0]],
                                  sem_s).wait()

        def half(i, slot):
            wait_load()
            start_scat(slot)

            @pl.when(jnp.logical_and(i >= 1, i + 1 < n_ch))
            def _():
                wait_scat()

            @pl.when(i + 1 < n_ch)
            def _():
                start_load(i + 1, 1 - slot)

        def it2(j, carry):
            half(2 * j, 0)
            half(2 * j + 1, 1)
            return carry

        start_load(0, 0)
        jax.lax.fori_loop(0, n_ch // 2, it2, 0)
        wait_scat()
        wait_scat()
        plsc.subcore_barrier()
        pltpu.sync_copy(hsh.at[pl.ds(s * rps, rps)],
                        hs_out.at[c, pl.ds(s * rps, rps)])
        pltpu.sync_copy(dsh.at[pl.ds(s * rps, rps)],
                        ds_out.at[c, pl.ds(s * rps, rps)])

    hsP, dsP = pl.kernel(
        body, out_type=out_type, mesh=_sc_mesh(), scratch_types=scratch,
        compiler_params=pltpu.CompilerParams(use_tc_tiling_on_sc=False),
    )(dst, wmsg, wex, zeros)
    return hsP[0], hsP[1], dsP[0], dsP[1]


def kernel(node_features, edge_features, params, edge_index, agent_index):
    p = params
    f32 = jnp.float32
    src = edge_index[0]
    dst = edge_index[1]

    M = _kron8(jnp.full((HID, HID), 1.0 / HID, f32))
    SX = _kron8(jnp.kron(jnp.eye(4, dtype=f32), jnp.ones((4, 4), f32)))

    WinA = jnp.zeros((HID, HID), f32).at[:NODE_IN].set(p['W_in'][:NODE_IN])
    WinB = jnp.zeros((HID, HID), f32).at[:AGENT_DIM].set(p['W_in'][NODE_IN:])
    nfp = jnp.pad(node_features, ((0, 0), (0, HID - NODE_IN)))
    afp = jnp.pad(p['agent_features'], ((0, 0), (0, HID - AGENT_DIM)))
    afg = _gather_agent(afp, agent_index)
    bt = _t8(p['b_in'] + p['b_pe'])

    h8, q08, k08, v08 = _tc_full(_prep_body, 4)(
        nfp.reshape(RN, 128), afg.reshape(RN, 128),
        p['positional_embedding'].reshape(RN, 128),
        _kron8(WinA), _kron8(WinB), _kron8(p['W_pe']), bt,
        _kron8(p['Wq'][0] * 0.5), _t8(p['bq'][0] * 0.5),
        _kron8(p['Wk'][0]), _t8(p['bk'][0]),
        _kron8(p['Wv'][0]), _t8(p['bv'][0]))

    NS = 1
    ES = E // NS
    RS = RE // NS
    qt0 = q08.reshape(N, HID)
    kt0 = k08.reshape(N, HID)
    vt0 = v08.reshape(N, HID)

    R = jnp.kron(jnp.eye(8, dtype=f32),
                 p['W_e_in'].astype(f32)).astype(jnp.bfloat16)
    c0_w = (R, _t8(p['b_e_in']),
            _kron8(p['We'][0]), _t8(p['be'][0]), SX,
            _kron8(p['Woe'][0]), _t8(p['boe'][0]),
            _kron8(p['Wef1'][0]), _t8(p['bef1'][0]),
            _kron8(p['Wef2'][0]), _t8(p['bef2'][0]),
            _t8(p['lne1_s'][0]), _t8(p['lne1_b'][0]),
            _t8(p['lne2_s'][0]), _t8(p['lne2_b'][0]), M)
    ef8 = edge_features.reshape(RE, 8)
    zeros = jnp.zeros((N // 16, HID), f32)

    g0 = [_sc_gather([qt0, kt0, vt0], [dst, src], [0, 1, 1], ES, 1000,
                     base=s * ES, fold01=True) for s in range(NS)]
    c0 = [_tc_edge(_c0_body, [8, 128, 128], [128, 128, 128], RS,
                   out_dtypes=[jnp.bfloat16, jnp.float32, jnp.float32])(
        ef8[s * RS:(s + 1) * RS], g0[s][0].reshape(RS, 128),
        g0[s][1].reshape(RS, 128), *c0_w)
        for s in range(NS)]
    sc = [_sc_scatter(dst, c0[s][1].reshape(ES, HID),
                      c0[s][2].reshape(ES, HID), zeros, ES, base=s * ES)
          for s in range(NS)]

    zRN = jnp.zeros((RN, 128), f32)
    hs_args = [sc[s][i].reshape(RN, 128) for s in range(NS) for i in (0, 1)]
    ds_args = [sc[s][i].reshape(RN, 128) for s in range(NS) for i in (2, 3)]
    hs_args += [zRN] * (4 - len(hs_args))
    ds_args += [zRN] * (4 - len(ds_args))
    q18, k18 = _tc_full(_n0_body, 2)(
        h8, *hs_args, *ds_args,
        _kron8(p['Wo'][0]), _t8(p['bo'][0]),
        _kron8(p['Wf1'][0]), _t8(p['bf1'][0]),
        _kron8(p['Wf2'][0]), _t8(p['bf2'][0]),
        _t8(p['ln1_s'][0]), _t8(p['ln1_b'][0]),
        _t8(p['ln2_s'][0]), _t8(p['ln2_b'][0]), M,
        _kron8(p['Wq'][1] * 0.5), _t8(p['bq'][1] * 0.5),
        _kron8(p['Wk'][1]), _t8(p['bk'][1]))

    Wout8 = jnp.kron(jnp.eye(8, dtype=f32),
                     p['W_out'].astype(f32)).astype(jnp.bfloat16)
    bout = jnp.tile(p['b_out'].astype(f32), 8).reshape(1, 8)
    c1_w = (_kron8(p['We'][1]), _t8(p['be'][1]),
            _kron8(p['Woe'][1]), _t8(p['boe'][1]),
            _kron8(p['Wef1'][1]), _t8(p['bef1'][1]),
            _kron8(p['Wef2'][1]), _t8(p['bef2'][1]),
            _t8(p['lne1_s'][1]), _t8(p['lne1_b'][1]),
            _t8(p['lne2_s'][1]), _t8(p['lne2_b'][1]), M,
            Wout8, bout)
    qt1 = q18.reshape(N, HID)
    kt1 = k18.reshape(N, HID)

    g1 = [_sc_gather([qt1, kt1], [dst, src], [0, 1], ES, 1000, base=s * ES,
                     fold01=True)
          for s in range(NS)]
    esc = [_tc_edge(_c1_body, [128, 128], [8], RS)(
        c0[s][0], g1[s][0].reshape(RS, 128), *c1_w)[0]
        for s in range(NS)]

    return jnp.concatenate(esc, axis=0).reshape(E)
